# Initial kernel scaffold; baseline (speedup 1.0000x reference)
import jax
import jax.numpy as jnp
from jax.experimental import pallas as pl


def kernel(x, v0, xx_r, xx_w, xx_k, xx_v, xx_a, xx_g, ww_b, ww_w1, ww_w2, aa_b, aa_w1, aa_w2, vv_b, vv_w1, vv_w2, gg_w1, gg_w2, kk_s, ka_s, rk_s, W_r, W_k, W_v, W_o, gn_w, gn_b):
    raise NotImplementedError("write your pallas kernel here")


# speedup vs baseline: 3.5694x; 3.5694x over previous
import functools

import jax
import jax.numpy as jnp
from jax.experimental import pallas as pl
from jax.experimental.pallas import tpu as pltpu

_B, _T, _C = 2, 1024, 2048
_N = 64
_H = _C // _N
_L = 64
_NC = _T // _L
_GN_EPS = 0.00064
_EPS2 = 1e-24
_BT = _B * _T

_MB1 = 128
_MB3 = 256

_f32 = jnp.float32
_bf16 = jnp.bfloat16


def _k1_body(x_ref, dx_ref, v0_ref, mix_ref,
             wr_ref, wk_ref, wv_ref,
             ww1_ref, ww2_ref, aa1_ref, aa2_ref,
             vv1_ref, vv2_ref, gg1_ref, gg2_ref,
             r_ref, wln_ref, k_ref, v_ref, a_ref, g_ref):
    x = x_ref[...]
    dx = dx_ref[...]
    mv = mix_ref[...]
    xr = (x + dx * mv[0:1, :]).astype(_bf16)
    xw = x + dx * mv[1:2, :]
    xk = (x + dx * mv[2:3, :]).astype(_bf16)
    xv = x + dx * mv[3:4, :]
    xa = x + dx * mv[4:5, :]
    xg = x + dx * mv[5:6, :]

    dot = functools.partial(jnp.dot, preferred_element_type=_f32)

    r_ref[...] = dot(xr, wr_ref[...])
    k_ref[...] = dot(xk, wk_ref[...])
    vb = dot(xv.astype(_bf16), wv_ref[...])

    wl = mv[6:7, :] + dot(jnp.tanh(dot(xw, ww1_ref[...])), ww2_ref[...])
    sp = jnp.maximum(-wl, 0.0) + jnp.log1p(jnp.exp(-jnp.abs(wl)))
    wln_ref[...] = -jnp.exp(-sp - 0.5)

    a_ref[...] = jax.nn.sigmoid(mv[7:8, :] + dot(dot(xa, aa1_ref[...]), aa2_ref[...]))
    sv = jax.nn.sigmoid(mv[8:9, :] + dot(dot(xv, vv1_ref[...]), vv2_ref[...]))
    v_ref[...] = vb + (v0_ref[...] - vb) * sv
    g_ref[...] = dot(jax.nn.sigmoid(dot(xg, gg1_ref[...])), gg2_ref[...])


def _k2_body(r_ref, wln_ref, k_ref, v_ref, a_ref,
             kks_ref, kas_ref, rks_ref, gnw_ref, gnb_ref,
             y_ref):
    kks = kks_ref[0, :]
    kas = kas_ref[0, :]
    rks = rks_ref[0, :]

    row = jax.lax.broadcasted_iota(jnp.int32, (_L, _L), 0)
    col = jax.lax.broadcasted_iota(jnp.int32, (_L, _L), 1)
    tri_incl = (row >= col).astype(_f32)
    strict = row > col
    eye = (row == col).astype(_f32)

    dot = functools.partial(jnp.dot, preferred_element_type=_f32)
    dot_nt = lambda p, q: jax.lax.dot_general(
        p, q, (((1,), (1,)), ((), ())), preferred_element_type=_f32)
    dot_tn = lambda p, q: jax.lax.dot_general(
        p, q, (((0,), (0,)), ((), ())), preferred_element_type=_f32)

    def chunk(i, St):
        sl = pl.ds(pl.multiple_of(i * _L, _L), _L)
        rc = r_ref[0, 0, sl, :]
        wc = wln_ref[0, 0, sl, :]
        kraw = k_ref[0, 0, sl, :]
        vc = v_ref[0, 0, sl, :]
        ac = a_ref[0, 0, sl, :]

        kkc = kraw * kks
        ss = jnp.sum(kkc * kkc, axis=-1, keepdims=True)
        kkc = kkc * jax.lax.rsqrt(jnp.maximum(ss, _EPS2))
        kfc = kraw * (1.0 + (ac - 1.0) * kas)
        bc = kkc * ac
        anc = -kkc

        clog = dot(tri_incl, wc)
        c_in = jnp.exp(clog)
        inv_c = jnp.exp(-clog)
        at = anc * jnp.exp(clog - wc)
        bt = bc * inv_c
        kt = kfc * inv_c
        rt = rc * c_in

        lhs = jnp.concatenate([at, rt], axis=0)
        rhs = jnp.concatenate([bt, kt], axis=0)
        G = dot_nt(lhs, rhs)
        m1 = jnp.where(strict, G[:_L, :_L], 0.0)
        m2 = jnp.where(strict, G[:_L, _L:], 0.0)
        rb = G[_L:, :_L] * tri_incl
        rk = G[_L:, _L:] * tri_incl

        Nm = m1
        P = eye + m1
        for _ in range(5):
            Nm = dot(Nm, Nm)
            P = P + dot(Nm, P)

        vs = jnp.concatenate([vc, St], axis=0)
        zrhs = dot(jnp.concatenate([m2, at], axis=1), vs)
        Z = dot(P, zrhs)

        O = dot(jnp.concatenate([rb, rk, rt], axis=1),
                jnp.concatenate([Z, vc, St], axis=0))
        y_ref[0, 0, sl, :] = O

        St = (St + dot_tn(jnp.concatenate([bt, kt], axis=0),
                          jnp.concatenate([Z, vc], axis=0))) * c_in[_L - 1][:, None]
        return St

    jax.lax.fori_loop(0, _NC, chunk, jnp.zeros((_N, _N), _f32))

    y = y_ref[0, 0, :, :]
    mu = jnp.mean(y, axis=-1, keepdims=True)
    yc = y - mu
    var = jnp.mean(yc * yc, axis=-1, keepdims=True)
    yn = yc * jax.lax.rsqrt(var + _GN_EPS) * gnw_ref[0, :] + gnb_ref[0, :]

    r = r_ref[0, 0, :, :]
    kraw = k_ref[0, 0, :, :]
    ac = a_ref[0, 0, :, :]
    kf = kraw * (1.0 + (ac - 1.0) * kas)
    bonus = jnp.sum(r * kf * rks, axis=-1, keepdims=True) * v_ref[0, 0, :, :]
    y_ref[0, 0, :, :] = yn + bonus


def _k3_body(y_ref, g_ref, wo_ref, o_ref):
    yg = (y_ref[...] * g_ref[...]).astype(_bf16)
    o_ref[...] = jnp.dot(yg, wo_ref[...], preferred_element_type=_f32)


def kernel(x, v0, xx_r, xx_w, xx_k, xx_v, xx_a, xx_g, ww_b, ww_w1, ww_w2,
           aa_b, aa_w1, aa_w2, vv_b, vv_w1, vv_w2, gg_w1, gg_w2,
           kk_s, ka_s, rk_s, W_r, W_k, W_v, W_o, gn_w, gn_b):
    xs = jnp.pad(x, ((0, 0), (1, 0), (0, 0)))[:, :_T, :]
    x2 = x.reshape(_BT, _C)
    dx2 = (xs - x).reshape(_BT, _C)
    v02 = v0.reshape(_BT, _C)

    mix = jnp.concatenate(
        [t.reshape(1, _C) for t in (xx_r, xx_w, xx_k, xx_v, xx_a, xx_g,
                                    ww_b, aa_b, vv_b)]
        + [jnp.zeros((7, _C), _f32)], axis=0)

    row_spec = pl.BlockSpec((_MB1, _C), lambda i: (i, 0))
    full = lambda s: pl.BlockSpec(s, lambda i: tuple(0 for _ in s))

    grid1 = _BT // _MB1
    r2, wln2, k2, v2, a2, g2 = pl.pallas_call(
        _k1_body,
        grid=(grid1,),
        in_specs=[row_spec, row_spec, row_spec, full((16, _C)),
                  full((_C, _C)), full((_C, _C)), full((_C, _C)),
                  full(ww_w1.shape), full(ww_w2.shape),
                  full(aa_w1.shape), full(aa_w2.shape),
                  full(vv_w1.shape), full(vv_w2.shape),
                  full(gg_w1.shape), full(gg_w2.shape)],
        out_specs=[row_spec] * 6,
        out_shape=[jax.ShapeDtypeStruct((_BT, _C), _f32)] * 6,
        compiler_params=pltpu.CompilerParams(
            dimension_semantics=("parallel",),
            vmem_limit_bytes=64 * 1024 * 1024),
    )(x2, dx2, v02, mix,
      W_r.astype(_bf16), W_k.astype(_bf16), W_v.astype(_bf16),
      ww_w1, ww_w2, aa_w1, aa_w2, vv_w1, vv_w2, gg_w1, gg_w2)

    hm = lambda t: t.reshape(_B, _T, _H, _N).transpose(0, 2, 1, 3)
    hvec = lambda t: t.reshape(_H, 1, _N)

    head_spec = pl.BlockSpec((1, 1, _T, _N), lambda i: (i // _H, i % _H, 0, 0))
    hvec_spec = pl.BlockSpec((1, 1, _N), lambda i: (i % _H, 0, 0))

    yH = pl.pallas_call(
        _k2_body,
        grid=(_B * _H,),
        in_specs=[head_spec] * 5 + [hvec_spec] * 5,
        out_specs=head_spec,
        out_shape=jax.ShapeDtypeStruct((_B, _H, _T, _N), _f32),
        compiler_params=pltpu.CompilerParams(
            dimension_semantics=("parallel",),
            vmem_limit_bytes=64 * 1024 * 1024),
    )(hm(r2), hm(wln2), hm(k2), hm(v2), hm(a2),
      hvec(kk_s), hvec(ka_s), hvec(rk_s), hvec(gn_w), hvec(gn_b))

    yF = yH.transpose(0, 2, 1, 3).reshape(_BT, _C)

    row3 = pl.BlockSpec((_MB3, _C), lambda i: (i, 0))
    out2 = pl.pallas_call(
        _k3_body,
        grid=(_BT // _MB3,),
        in_specs=[row3, row3, full((_C, _C))],
        out_specs=row3,
        out_shape=jax.ShapeDtypeStruct((_BT, _C), _f32),
        compiler_params=pltpu.CompilerParams(
            dimension_semantics=("parallel",),
            vmem_limit_bytes=64 * 1024 * 1024),
    )(yF, g2, W_o.astype(_bf16))

    return out2.reshape(_B, _T, _C), v0


# speedup vs baseline: 3.7826x; 1.0597x over previous
import functools

import jax
import jax.numpy as jnp
from jax.experimental import pallas as pl
from jax.experimental.pallas import tpu as pltpu

_B, _T, _C = 2, 1024, 2048
_N = 64
_H = _C // _N
_L = 64
_NC = _T // _L
_GN_EPS = 0.00064
_EPS2 = 1e-24
_BT = _B * _T

_MB1 = 128
_MB3 = 256

_f32 = jnp.float32
_bf16 = jnp.bfloat16


def _k1_body(x_ref, dx_ref, v0_ref, mix_ref,
             wr_ref, wk_ref, wv_ref,
             ww1_ref, ww2_ref, aa1_ref, aa2_ref,
             vv1_ref, vv2_ref, gg1_ref, gg2_ref,
             r_ref, wln_ref, k_ref, v_ref, a_ref, g_ref):
    x = x_ref[...]
    dx = dx_ref[...]
    mv = mix_ref[...]
    xr = (x + dx * mv[0:1, :]).astype(_bf16)
    xw = x + dx * mv[1:2, :]
    xk = (x + dx * mv[2:3, :]).astype(_bf16)
    xv = x + dx * mv[3:4, :]
    xa = x + dx * mv[4:5, :]
    xg = x + dx * mv[5:6, :]

    dot = functools.partial(jnp.dot, preferred_element_type=_f32)

    r_ref[...] = dot(xr, wr_ref[...])
    k_ref[...] = dot(xk, wk_ref[...])
    vb = dot(xv.astype(_bf16), wv_ref[...])

    wl = mv[6:7, :] + dot(jnp.tanh(dot(xw, ww1_ref[...])), ww2_ref[...])
    sp = jnp.maximum(-wl, 0.0) + jnp.log1p(jnp.exp(-jnp.abs(wl)))
    wln_ref[...] = -jnp.exp(-sp - 0.5)

    a_ref[...] = jax.nn.sigmoid(mv[7:8, :] + dot(dot(xa, aa1_ref[...]), aa2_ref[...]))
    sv = jax.nn.sigmoid(mv[8:9, :] + dot(dot(xv, vv1_ref[...]), vv2_ref[...]))
    v_ref[...] = vb + (v0_ref[...] - vb) * sv
    g_ref[...] = dot(jax.nn.sigmoid(dot(xg, gg1_ref[...])), gg2_ref[...])


_HG = 4


def _k2_body(r_ref, wln_ref, k_ref, v_ref, a_ref,
             kks_ref, kas_ref, rks_ref, gnw_ref, gnb_ref,
             y_ref):
    row = jax.lax.broadcasted_iota(jnp.int32, (_L, _L), 0)
    col = jax.lax.broadcasted_iota(jnp.int32, (_L, _L), 1)
    tri_incl = (row >= col).astype(_f32)
    strict = row > col
    eye = (row == col).astype(_f32)

    dot = functools.partial(jnp.dot, preferred_element_type=_f32)
    dot_nt = lambda p, q: jax.lax.dot_general(
        p, q, (((1,), (1,)), ((), ())), preferred_element_type=_f32)
    dot_tn = lambda p, q: jax.lax.dot_general(
        p, q, (((0,), (0,)), ((), ())), preferred_element_type=_f32)

    def chunk(i, Sts):
        sl = pl.ds(pl.multiple_of(i * _L, _L), _L)
        new_Sts = []
        for h in range(_HG):
            St = Sts[h]
            kks = kks_ref[h, :]
            kas = kas_ref[h, :]
            rc = r_ref[0, h, sl, :]
            wc = wln_ref[0, h, sl, :]
            kraw = k_ref[0, h, sl, :]
            vc = v_ref[0, h, sl, :]
            ac = a_ref[0, h, sl, :]

            kkc = kraw * kks
            ss = jnp.sum(kkc * kkc, axis=-1, keepdims=True)
            kkc = kkc * jax.lax.rsqrt(jnp.maximum(ss, _EPS2))
            kfc = kraw * (1.0 + (ac - 1.0) * kas)
            bc = kkc * ac
            anc = -kkc

            clog = dot(tri_incl, wc)
            c_in = jnp.exp(clog)
            inv_c = jnp.exp(-clog)
            at = anc * jnp.exp(clog - wc)
            bt = bc * inv_c
            kt = kfc * inv_c
            rt = rc * c_in

            lhs = jnp.concatenate([at, rt], axis=0)
            rhs = jnp.concatenate([bt, kt], axis=0)
            G = dot_nt(lhs, rhs)
            m1 = jnp.where(strict, G[:_L, :_L], 0.0)
            m2 = jnp.where(strict, G[:_L, _L:], 0.0)
            rb = G[_L:, :_L] * tri_incl
            rk = G[_L:, _L:] * tri_incl

            Nm = m1
            P = eye + m1
            for _ in range(5):
                Nm = dot(Nm, Nm)
                P = P + dot(Nm, P)

            vs = jnp.concatenate([vc, St], axis=0)
            zrhs = dot(jnp.concatenate([m2, at], axis=1), vs)
            Z = dot(P, zrhs)

            O = dot(jnp.concatenate([rb, rk, rt], axis=1),
                    jnp.concatenate([Z, vc, St], axis=0))
            y_ref[0, h, sl, :] = O

            St = (St + dot_tn(jnp.concatenate([bt, kt], axis=0),
                              jnp.concatenate([Z, vc], axis=0))) * c_in[_L - 1][:, None]
            new_Sts.append(St)
        return tuple(new_Sts)

    jax.lax.fori_loop(0, _NC, chunk,
                      tuple(jnp.zeros((_N, _N), _f32) for _ in range(_HG)))

    for h in range(_HG):
        kas = kas_ref[h, :]
        rks = rks_ref[h, :]
        y = y_ref[0, h, :, :]
        mu = jnp.mean(y, axis=-1, keepdims=True)
        yc = y - mu
        var = jnp.mean(yc * yc, axis=-1, keepdims=True)
        yn = yc * jax.lax.rsqrt(var + _GN_EPS) * gnw_ref[h, :] + gnb_ref[h, :]

        r = r_ref[0, h, :, :]
        kraw = k_ref[0, h, :, :]
        ac = a_ref[0, h, :, :]
        kf = kraw * (1.0 + (ac - 1.0) * kas)
        bonus = jnp.sum(r * kf * rks, axis=-1, keepdims=True) * v_ref[0, h, :, :]
        y_ref[0, h, :, :] = yn + bonus


def _k3_body(y_ref, g_ref, wo_ref, o_ref):
    yg = (y_ref[...] * g_ref[...]).astype(_bf16)
    o_ref[...] = jnp.dot(yg, wo_ref[...], preferred_element_type=_f32)


def kernel(x, v0, xx_r, xx_w, xx_k, xx_v, xx_a, xx_g, ww_b, ww_w1, ww_w2,
           aa_b, aa_w1, aa_w2, vv_b, vv_w1, vv_w2, gg_w1, gg_w2,
           kk_s, ka_s, rk_s, W_r, W_k, W_v, W_o, gn_w, gn_b):
    xs = jnp.pad(x, ((0, 0), (1, 0), (0, 0)))[:, :_T, :]
    x2 = x.reshape(_BT, _C)
    dx2 = (xs - x).reshape(_BT, _C)
    v02 = v0.reshape(_BT, _C)

    mix = jnp.concatenate(
        [t.reshape(1, _C) for t in (xx_r, xx_w, xx_k, xx_v, xx_a, xx_g,
                                    ww_b, aa_b, vv_b)]
        + [jnp.zeros((7, _C), _f32)], axis=0)

    row_spec = pl.BlockSpec((_MB1, _C), lambda i: (i, 0))
    full = lambda s: pl.BlockSpec(s, lambda i: tuple(0 for _ in s))

    grid1 = _BT // _MB1
    r2, wln2, k2, v2, a2, g2 = pl.pallas_call(
        _k1_body,
        grid=(grid1,),
        in_specs=[row_spec, row_spec, row_spec, full((16, _C)),
                  full((_C, _C)), full((_C, _C)), full((_C, _C)),
                  full(ww_w1.shape), full(ww_w2.shape),
                  full(aa_w1.shape), full(aa_w2.shape),
                  full(vv_w1.shape), full(vv_w2.shape),
                  full(gg_w1.shape), full(gg_w2.shape)],
        out_specs=[row_spec] * 6,
        out_shape=[jax.ShapeDtypeStruct((_BT, _C), _f32)] * 6,
        compiler_params=pltpu.CompilerParams(
            dimension_semantics=("parallel",),
            vmem_limit_bytes=64 * 1024 * 1024),
    )(x2, dx2, v02, mix,
      W_r.astype(_bf16), W_k.astype(_bf16), W_v.astype(_bf16),
      ww_w1, ww_w2, aa_w1, aa_w2, vv_w1, vv_w2, gg_w1, gg_w2)

    hm = lambda t: t.reshape(_B, _T, _H, _N).transpose(0, 2, 1, 3)
    hvec = lambda t: t.reshape(_H, 1, _N)

    ng = _H // _HG
    head_spec = pl.BlockSpec((1, _HG, _T, _N), lambda i: (i // ng, i % ng, 0, 0))
    hvec_spec = pl.BlockSpec((_HG, 1, _N), lambda i: (i % ng, 0, 0))

    yH = pl.pallas_call(
        _k2_body,
        grid=(_B * ng,),
        in_specs=[head_spec] * 5 + [hvec_spec] * 5,
        out_specs=head_spec,
        out_shape=jax.ShapeDtypeStruct((_B, _H, _T, _N), _f32),
        compiler_params=pltpu.CompilerParams(
            dimension_semantics=("parallel",),
            vmem_limit_bytes=64 * 1024 * 1024),
    )(hm(r2), hm(wln2), hm(k2), hm(v2), hm(a2),
      hvec(kk_s), hvec(ka_s), hvec(rk_s), hvec(gn_w), hvec(gn_b))

    yF = yH.transpose(0, 2, 1, 3).reshape(_BT, _C)

    row3 = pl.BlockSpec((_MB3, _C), lambda i: (i, 0))
    out2 = pl.pallas_call(
        _k3_body,
        grid=(_BT // _MB3,),
        in_specs=[row3, row3, full((_C, _C))],
        out_specs=row3,
        out_shape=jax.ShapeDtypeStruct((_BT, _C), _f32),
        compiler_params=pltpu.CompilerParams(
            dimension_semantics=("parallel",),
            vmem_limit_bytes=64 * 1024 * 1024),
    )(yF, g2, W_o.astype(_bf16))

    return out2.reshape(_B, _T, _C), v0


# speedup vs baseline: 3.9485x; 1.0439x over previous
import functools

import jax
import jax.numpy as jnp
from jax.experimental import pallas as pl
from jax.experimental.pallas import tpu as pltpu

_B, _T, _C = 2, 1024, 2048
_N = 64
_H = _C // _N
_L = 64
_NC = _T // _L
_GN_EPS = 0.00064
_EPS2 = 1e-24
_BT = _B * _T

_MB1 = 128
_MB3 = 256

_f32 = jnp.float32
_bf16 = jnp.bfloat16


def _k1_body(x_ref, dx_ref, v0_ref, mix_ref,
             wr_ref, wk_ref, wv_ref,
             ww1_ref, ww2_ref, aa1_ref, aa2_ref,
             vv1_ref, vv2_ref, gg1_ref, gg2_ref,
             r_ref, wln_ref, k_ref, v_ref, a_ref, g_ref):
    x = x_ref[...]
    dx = dx_ref[...]
    mv = mix_ref[...]
    xr = (x + dx * mv[0:1, :]).astype(_bf16)
    xw = x + dx * mv[1:2, :]
    xk = (x + dx * mv[2:3, :]).astype(_bf16)
    xv = x + dx * mv[3:4, :]
    xa = x + dx * mv[4:5, :]
    xg = x + dx * mv[5:6, :]

    dot = functools.partial(jnp.dot, preferred_element_type=_f32)

    r_ref[...] = dot(xr, wr_ref[...]).astype(_bf16)
    k_ref[...] = dot(xk, wk_ref[...]).astype(_bf16)
    vb = dot(xv.astype(_bf16), wv_ref[...])

    wl = mv[6:7, :] + dot(jnp.tanh(dot(xw, ww1_ref[...])), ww2_ref[...])
    sp = jnp.maximum(-wl, 0.0) + jnp.log1p(jnp.exp(-jnp.abs(wl)))
    wln_ref[...] = -jnp.exp(-sp - 0.5)

    a_ref[...] = jax.nn.sigmoid(
        mv[7:8, :] + dot(dot(xa, aa1_ref[...]), aa2_ref[...])).astype(_bf16)
    sv = jax.nn.sigmoid(mv[8:9, :] + dot(dot(xv, vv1_ref[...]), vv2_ref[...]))
    v_ref[...] = (vb + (v0_ref[...] - vb) * sv).astype(_bf16)
    g_ref[...] = dot(jax.nn.sigmoid(dot(xg, gg1_ref[...])), gg2_ref[...]).astype(_bf16)


_HG = 4


def _k2_body(r_ref, wln_ref, k_ref, v_ref, a_ref,
             kks_ref, kas_ref, rks_ref, gnw_ref, gnb_ref,
             y_ref):
    row = jax.lax.broadcasted_iota(jnp.int32, (_L, _L), 0)
    col = jax.lax.broadcasted_iota(jnp.int32, (_L, _L), 1)
    tri_incl = (row >= col).astype(_f32)
    strict = row > col
    eye = (row == col).astype(_f32)

    dot = functools.partial(jnp.dot, preferred_element_type=_f32)
    dot_nt = lambda p, q: jax.lax.dot_general(
        p, q, (((1,), (1,)), ((), ())), preferred_element_type=_f32)
    dot_tn = lambda p, q: jax.lax.dot_general(
        p, q, (((0,), (0,)), ((), ())), preferred_element_type=_f32)

    def chunk(i, Sts):
        sl = pl.ds(pl.multiple_of(i * _L, _L), _L)
        new_Sts = []
        for h in range(_HG):
            St = Sts[h]
            kks = kks_ref[h, :]
            kas = kas_ref[h, :]
            rc = r_ref[0, h, sl, :].astype(_f32)
            wc = wln_ref[0, h, sl, :]
            kraw = k_ref[0, h, sl, :].astype(_f32)
            vc = v_ref[0, h, sl, :].astype(_f32)
            ac = a_ref[0, h, sl, :].astype(_f32)

            kkc = kraw * kks
            ss = jnp.sum(kkc * kkc, axis=-1, keepdims=True)
            kkc = kkc * jax.lax.rsqrt(jnp.maximum(ss, _EPS2))
            kfc = kraw * (1.0 + (ac - 1.0) * kas)
            bc = kkc * ac
            anc = -kkc

            clog = dot(tri_incl, wc)
            c_in = jnp.exp(clog)
            inv_c = jnp.exp(-clog)
            at = anc * jnp.exp(clog - wc)
            bt = bc * inv_c
            kt = kfc * inv_c
            rt = rc * c_in

            lhs = jnp.concatenate([at, rt], axis=0)
            rhs = jnp.concatenate([bt, kt], axis=0)
            G = dot_nt(lhs, rhs)
            m1 = jnp.where(strict, G[:_L, :_L], 0.0)
            m2 = jnp.where(strict, G[:_L, _L:], 0.0)
            rb = G[_L:, :_L] * tri_incl
            rk = G[_L:, _L:] * tri_incl

            Nm = m1
            P = eye + m1
            for _ in range(5):
                Nm = dot(Nm, Nm)
                P = P + dot(Nm, P)

            vs = jnp.concatenate([vc, St], axis=0)
            zrhs = dot(jnp.concatenate([m2, at], axis=1), vs)
            Z = dot(P, zrhs)

            O = dot(jnp.concatenate([rb, rk, rt], axis=1),
                    jnp.concatenate([Z, vc, St], axis=0))
            y_ref[0, h, sl, :] = O.astype(_bf16)

            St = (St + dot_tn(jnp.concatenate([bt, kt], axis=0),
                              jnp.concatenate([Z, vc], axis=0))) * c_in[_L - 1][:, None]
            new_Sts.append(St)
        return tuple(new_Sts)

    jax.lax.fori_loop(0, _NC, chunk,
                      tuple(jnp.zeros((_N, _N), _f32) for _ in range(_HG)))

    for h in range(_HG):
        kas = kas_ref[h, :]
        rks = rks_ref[h, :]
        y = y_ref[0, h, :, :].astype(_f32)
        mu = jnp.mean(y, axis=-1, keepdims=True)
        yc = y - mu
        var = jnp.mean(yc * yc, axis=-1, keepdims=True)
        yn = yc * jax.lax.rsqrt(var + _GN_EPS) * gnw_ref[h, :] + gnb_ref[h, :]

        r = r_ref[0, h, :, :].astype(_f32)
        kraw = k_ref[0, h, :, :].astype(_f32)
        ac = a_ref[0, h, :, :].astype(_f32)
        kf = kraw * (1.0 + (ac - 1.0) * kas)
        bonus = jnp.sum(r * kf * rks, axis=-1, keepdims=True) * v_ref[0, h, :, :].astype(_f32)
        y_ref[0, h, :, :] = (yn + bonus).astype(_bf16)


def _k3_body(y_ref, g_ref, wo_ref, o_ref):
    yg = y_ref[...] * g_ref[...]
    o_ref[...] = jnp.dot(yg, wo_ref[...], preferred_element_type=_f32)


def kernel(x, v0, xx_r, xx_w, xx_k, xx_v, xx_a, xx_g, ww_b, ww_w1, ww_w2,
           aa_b, aa_w1, aa_w2, vv_b, vv_w1, vv_w2, gg_w1, gg_w2,
           kk_s, ka_s, rk_s, W_r, W_k, W_v, W_o, gn_w, gn_b):
    xs = jnp.pad(x, ((0, 0), (1, 0), (0, 0)))[:, :_T, :]
    x2 = x.reshape(_BT, _C)
    dx2 = (xs - x).reshape(_BT, _C)
    v02 = v0.reshape(_BT, _C)

    mix = jnp.concatenate(
        [t.reshape(1, _C) for t in (xx_r, xx_w, xx_k, xx_v, xx_a, xx_g,
                                    ww_b, aa_b, vv_b)]
        + [jnp.zeros((7, _C), _f32)], axis=0)

    row_spec = pl.BlockSpec((_MB1, _C), lambda i: (i, 0))
    full = lambda s: pl.BlockSpec(s, lambda i: tuple(0 for _ in s))

    grid1 = _BT // _MB1
    r2, wln2, k2, v2, a2, g2 = pl.pallas_call(
        _k1_body,
        grid=(grid1,),
        in_specs=[row_spec, row_spec, row_spec, full((16, _C)),
                  full((_C, _C)), full((_C, _C)), full((_C, _C)),
                  full(ww_w1.shape), full(ww_w2.shape),
                  full(aa_w1.shape), full(aa_w2.shape),
                  full(vv_w1.shape), full(vv_w2.shape),
                  full(gg_w1.shape), full(gg_w2.shape)],
        out_specs=[row_spec] * 6,
        out_shape=[jax.ShapeDtypeStruct((_BT, _C), _bf16),
                   jax.ShapeDtypeStruct((_BT, _C), _f32),
                   jax.ShapeDtypeStruct((_BT, _C), _bf16),
                   jax.ShapeDtypeStruct((_BT, _C), _bf16),
                   jax.ShapeDtypeStruct((_BT, _C), _bf16),
                   jax.ShapeDtypeStruct((_BT, _C), _bf16)],
        compiler_params=pltpu.CompilerParams(
            dimension_semantics=("parallel",),
            vmem_limit_bytes=64 * 1024 * 1024),
    )(x2, dx2, v02, mix,
      W_r.astype(_bf16), W_k.astype(_bf16), W_v.astype(_bf16),
      ww_w1, ww_w2, aa_w1, aa_w2, vv_w1, vv_w2, gg_w1, gg_w2)

    hm = lambda t: t.reshape(_B, _T, _H, _N).transpose(0, 2, 1, 3)
    hvec = lambda t: t.reshape(_H, 1, _N)

    ng = _H // _HG
    head_spec = pl.BlockSpec((1, _HG, _T, _N), lambda i: (i // ng, i % ng, 0, 0))
    hvec_spec = pl.BlockSpec((_HG, 1, _N), lambda i: (i % ng, 0, 0))

    yH = pl.pallas_call(
        _k2_body,
        grid=(_B * ng,),
        in_specs=[head_spec] * 5 + [hvec_spec] * 5,
        out_specs=head_spec,
        out_shape=jax.ShapeDtypeStruct((_B, _H, _T, _N), _bf16),
        compiler_params=pltpu.CompilerParams(
            dimension_semantics=("parallel",),
            vmem_limit_bytes=64 * 1024 * 1024),
    )(hm(r2), hm(wln2), hm(k2), hm(v2), hm(a2),
      hvec(kk_s), hvec(ka_s), hvec(rk_s), hvec(gn_w), hvec(gn_b))

    yF = yH.transpose(0, 2, 1, 3).reshape(_BT, _C)

    row3 = pl.BlockSpec((_MB3, _C), lambda i: (i, 0))
    out2 = pl.pallas_call(
        _k3_body,
        grid=(_BT // _MB3,),
        in_specs=[row3, row3, full((_C, _C))],
        out_specs=row3,
        out_shape=jax.ShapeDtypeStruct((_BT, _C), _f32),
        compiler_params=pltpu.CompilerParams(
            dimension_semantics=("parallel",),
            vmem_limit_bytes=64 * 1024 * 1024),
    )(yF, g2, W_o.astype(_bf16))

    return out2.reshape(_B, _T, _C), v0


# speedup vs baseline: 6.3333x; 1.6040x over previous
import functools

import jax
import jax.numpy as jnp
from jax.experimental import pallas as pl
from jax.experimental.pallas import tpu as pltpu

_B, _T, _C = 2, 1024, 2048
_N = 64
_H = _C // _N
_L = 64
_NC = _T // _L
_GN_EPS = 0.00064
_EPS2 = 1e-24
_BT = _B * _T

_MB1 = 128
_MB3 = 256

_f32 = jnp.float32
_bf16 = jnp.bfloat16


def _k1_body(x_ref, dx_ref, v0_ref, mix_ref,
             wr_ref, wk_ref, wv_ref,
             ww1_ref, ww2_ref, aa1_ref, aa2_ref,
             vv1_ref, vv2_ref, gg1_ref, gg2_ref,
             r_ref, wln_ref, k_ref, v_ref, a_ref, g_ref):
    x = x_ref[...]
    dx = dx_ref[...]
    mv = mix_ref[...]
    xr = (x + dx * mv[0:1, :]).astype(_bf16)
    xw = x + dx * mv[1:2, :]
    xk = (x + dx * mv[2:3, :]).astype(_bf16)
    xv = x + dx * mv[3:4, :]
    xa = x + dx * mv[4:5, :]
    xg = x + dx * mv[5:6, :]

    dot = functools.partial(jnp.dot, preferred_element_type=_f32)

    r_ref[...] = dot(xr, wr_ref[...]).astype(_bf16)
    k_ref[...] = dot(xk, wk_ref[...]).astype(_bf16)
    vb = dot(xv.astype(_bf16), wv_ref[...])

    wl = mv[6:7, :] + dot(jnp.tanh(dot(xw, ww1_ref[...])), ww2_ref[...])
    sp = jnp.maximum(-wl, 0.0) + jnp.log1p(jnp.exp(-jnp.abs(wl)))
    wln_ref[...] = -jnp.exp(-sp - 0.5)

    a_ref[...] = jax.nn.sigmoid(
        mv[7:8, :] + dot(dot(xa, aa1_ref[...]), aa2_ref[...])).astype(_bf16)
    sv = jax.nn.sigmoid(mv[8:9, :] + dot(dot(xv, vv1_ref[...]), vv2_ref[...]))
    v_ref[...] = (vb + (v0_ref[...] - vb) * sv).astype(_bf16)
    g_ref[...] = dot(jax.nn.sigmoid(dot(xg, gg1_ref[...])), gg2_ref[...]).astype(_bf16)


_HG = 4
_GL = _HG * _L


def _k2_body(r_ref, wln_ref, k_ref, v_ref, a_ref,
             kks_ref, kas_ref, rks_ref, gnw_ref, gnb_ref,
             y_ref):
    row = jax.lax.broadcasted_iota(jnp.int32, (_GL, _GL), 0)
    col = jax.lax.broadcasted_iota(jnp.int32, (_GL, _GL), 1)
    same = (row >> 6) == (col >> 6)
    rpos = row & 63
    cpos = col & 63
    m_strict = same & (rpos > cpos)
    m_incl = (same & (rpos >= cpos)).astype(_f32)
    eye_g = (row == col).astype(_f32)
    sel_last = (col == (row | 63)).astype(_f32)
    lrow = jax.lax.broadcasted_iota(jnp.int32, (_GL, _N), 0)
    lcol = jax.lax.broadcasted_iota(jnp.int32, (_GL, _N), 1)
    lane_diag = (lcol == (lrow & 63)).astype(_f32)
    same_f = same.astype(_f32)

    dot = functools.partial(jnp.dot, preferred_element_type=_f32)
    dot_nt = lambda p, q: jax.lax.dot_general(
        p, q, (((1,), (1,)), ((), ())), preferred_element_type=_f32)
    dot_tn = lambda p, q: jax.lax.dot_general(
        p, q, (((0,), (0,)), ((), ())), preferred_element_type=_f32)
    bdiag = lambda x: jnp.concatenate([x] * _HG, axis=1) * same_f

    ones_L = jnp.ones((_L, 1), _f32)
    kks_s = jnp.concatenate([ones_L * kks_ref[h, :] for h in range(_HG)], 0)
    kas_s = jnp.concatenate([ones_L * kas_ref[h, :] for h in range(_HG)], 0)

    def chunk(i, St):
        sl = pl.ds(pl.multiple_of(i * _L, _L), _L)
        cat = lambda ref: jnp.concatenate(
            [ref[0, h, sl, :] for h in range(_HG)], axis=0)
        rc = cat(r_ref).astype(_f32)
        wc = cat(wln_ref)
        kraw = cat(k_ref).astype(_f32)
        vc = cat(v_ref).astype(_f32)
        ac = cat(a_ref).astype(_f32)

        kkc = kraw * kks_s
        ss = jnp.sum(kkc * kkc, axis=-1, keepdims=True)
        kkc = kkc * jax.lax.rsqrt(jnp.maximum(ss, _EPS2))
        kfc = kraw * (1.0 + (ac - 1.0) * kas_s)
        bc = kkc * ac
        anc = -kkc

        clog = dot(m_incl, wc)
        c_in = jnp.exp(clog)
        inv_c = jnp.exp(-clog)
        at = anc * jnp.exp(clog - wc)
        bt = bc * inv_c
        kt = kfc * inv_c
        rt = rc * c_in

        lhs = jnp.concatenate([at, rt], axis=0)
        rhs = jnp.concatenate([bt, kt], axis=0)
        G = dot_nt(lhs, rhs)
        m1 = jnp.where(m_strict, G[:_GL, :_GL], 0.0)
        m2 = jnp.where(m_strict, G[:_GL, _GL:], 0.0)
        rb = G[_GL:, :_GL] * m_incl
        rk = G[_GL:, _GL:] * m_incl

        Nm = m1
        P = eye_g + m1
        for _ in range(5):
            Nm = dot(Nm, Nm)
            P = P + dot(Nm, P)

        zrhs = dot(jnp.concatenate([m2, bdiag(at)], axis=1),
                   jnp.concatenate([vc, St], axis=0))
        Z = dot(P, zrhs)

        O = dot(jnp.concatenate([rb, bdiag(rt), rk], axis=1),
                jnp.concatenate([Z, St, vc], axis=0))
        for h in range(_HG):
            y_ref[0, h, sl, :] = O[h * _L:(h + 1) * _L, :].astype(_bf16)

        delta = dot_tn(jnp.concatenate([bdiag(bt), bdiag(kt)], axis=0),
                       jnp.concatenate([Z, vc], axis=0))
        scale = jnp.sum(dot(sel_last, c_in) * lane_diag, axis=-1, keepdims=True)
        return (St + delta) * scale

    jax.lax.fori_loop(0, _NC, chunk, jnp.zeros((_GL, _N), _f32))

    for h in range(_HG):
        kas = kas_ref[h, :]
        rks = rks_ref[h, :]
        y = y_ref[0, h, :, :].astype(_f32)
        mu = jnp.mean(y, axis=-1, keepdims=True)
        yc = y - mu
        var = jnp.mean(yc * yc, axis=-1, keepdims=True)
        yn = yc * jax.lax.rsqrt(var + _GN_EPS) * gnw_ref[h, :] + gnb_ref[h, :]

        r = r_ref[0, h, :, :].astype(_f32)
        kraw = k_ref[0, h, :, :].astype(_f32)
        ac = a_ref[0, h, :, :].astype(_f32)
        kf = kraw * (1.0 + (ac - 1.0) * kas)
        bonus = jnp.sum(r * kf * rks, axis=-1, keepdims=True) * v_ref[0, h, :, :].astype(_f32)
        y_ref[0, h, :, :] = (yn + bonus).astype(_bf16)


def _k3_body(y_ref, g_ref, wo_ref, o_ref):
    yg = y_ref[...] * g_ref[...]
    o_ref[...] = jnp.dot(yg, wo_ref[...], preferred_element_type=_f32)


def kernel(x, v0, xx_r, xx_w, xx_k, xx_v, xx_a, xx_g, ww_b, ww_w1, ww_w2,
           aa_b, aa_w1, aa_w2, vv_b, vv_w1, vv_w2, gg_w1, gg_w2,
           kk_s, ka_s, rk_s, W_r, W_k, W_v, W_o, gn_w, gn_b):
    xs = jnp.pad(x, ((0, 0), (1, 0), (0, 0)))[:, :_T, :]
    x2 = x.reshape(_BT, _C)
    dx2 = (xs - x).reshape(_BT, _C)
    v02 = v0.reshape(_BT, _C)

    mix = jnp.concatenate(
        [t.reshape(1, _C) for t in (xx_r, xx_w, xx_k, xx_v, xx_a, xx_g,
                                    ww_b, aa_b, vv_b)]
        + [jnp.zeros((7, _C), _f32)], axis=0)

    row_spec = pl.BlockSpec((_MB1, _C), lambda i: (i, 0))
    full = lambda s: pl.BlockSpec(s, lambda i: tuple(0 for _ in s))

    grid1 = _BT // _MB1
    r2, wln2, k2, v2, a2, g2 = pl.pallas_call(
        _k1_body,
        grid=(grid1,),
        in_specs=[row_spec, row_spec, row_spec, full((16, _C)),
                  full((_C, _C)), full((_C, _C)), full((_C, _C)),
                  full(ww_w1.shape), full(ww_w2.shape),
                  full(aa_w1.shape), full(aa_w2.shape),
                  full(vv_w1.shape), full(vv_w2.shape),
                  full(gg_w1.shape), full(gg_w2.shape)],
        out_specs=[row_spec] * 6,
        out_shape=[jax.ShapeDtypeStruct((_BT, _C), _bf16),
                   jax.ShapeDtypeStruct((_BT, _C), _f32),
                   jax.ShapeDtypeStruct((_BT, _C), _bf16),
                   jax.ShapeDtypeStruct((_BT, _C), _bf16),
                   jax.ShapeDtypeStruct((_BT, _C), _bf16),
                   jax.ShapeDtypeStruct((_BT, _C), _bf16)],
        compiler_params=pltpu.CompilerParams(
            dimension_semantics=("parallel",),
            vmem_limit_bytes=64 * 1024 * 1024),
    )(x2, dx2, v02, mix,
      W_r.astype(_bf16), W_k.astype(_bf16), W_v.astype(_bf16),
      ww_w1, ww_w2, aa_w1, aa_w2, vv_w1, vv_w2, gg_w1, gg_w2)

    hm = lambda t: t.reshape(_B, _T, _H, _N).transpose(0, 2, 1, 3)
    hvec = lambda t: t.reshape(_H, 1, _N)

    ng = _H // _HG
    head_spec = pl.BlockSpec((1, _HG, _T, _N), lambda i: (i // ng, i % ng, 0, 0))
    hvec_spec = pl.BlockSpec((_HG, 1, _N), lambda i: (i % ng, 0, 0))

    yH = pl.pallas_call(
        _k2_body,
        grid=(_B * ng,),
        in_specs=[head_spec] * 5 + [hvec_spec] * 5,
        out_specs=head_spec,
        out_shape=jax.ShapeDtypeStruct((_B, _H, _T, _N), _bf16),
        compiler_params=pltpu.CompilerParams(
            dimension_semantics=("parallel",),
            vmem_limit_bytes=64 * 1024 * 1024),
    )(hm(r2), hm(wln2), hm(k2), hm(v2), hm(a2),
      hvec(kk_s), hvec(ka_s), hvec(rk_s), hvec(gn_w), hvec(gn_b))

    yF = yH.transpose(0, 2, 1, 3).reshape(_BT, _C)

    row3 = pl.BlockSpec((_MB3, _C), lambda i: (i, 0))
    out2 = pl.pallas_call(
        _k3_body,
        grid=(_BT // _MB3,),
        in_specs=[row3, row3, full((_C, _C))],
        out_specs=row3,
        out_shape=jax.ShapeDtypeStruct((_BT, _C), _f32),
        compiler_params=pltpu.CompilerParams(
            dimension_semantics=("parallel",),
            vmem_limit_bytes=64 * 1024 * 1024),
    )(yF, g2, W_o.astype(_bf16))

    return out2.reshape(_B, _T, _C), v0


# speedup vs baseline: 7.5171x; 1.1869x over previous
import functools

import jax
import jax.numpy as jnp
from jax.experimental import pallas as pl
from jax.experimental.pallas import tpu as pltpu

_B, _T, _C = 2, 1024, 2048
_N = 64
_H = _C // _N
_L = 64
_NC = _T // _L
_GN_EPS = 0.00064
_EPS2 = 1e-24
_BT = _B * _T

_MB1 = 128
_MB3 = 256

_f32 = jnp.float32
_bf16 = jnp.bfloat16


def _k1_body(x_ref, dx_ref, v0_ref, mix_ref,
             wr_ref, wk_ref, wv_ref,
             ww1_ref, ww2_ref, aa1_ref, aa2_ref,
             vv1_ref, vv2_ref, gg1_ref, gg2_ref,
             r_ref, wln_ref, k_ref, v_ref, a_ref, g_ref):
    x = x_ref[...]
    dx = dx_ref[...]
    mv = mix_ref[...]
    xr = (x + dx * mv[0:1, :]).astype(_bf16)
    xw = x + dx * mv[1:2, :]
    xk = (x + dx * mv[2:3, :]).astype(_bf16)
    xv = x + dx * mv[3:4, :]
    xa = x + dx * mv[4:5, :]
    xg = x + dx * mv[5:6, :]

    dot = functools.partial(jnp.dot, preferred_element_type=_f32)

    r_ref[...] = dot(xr, wr_ref[...]).astype(_bf16)
    k_ref[...] = dot(xk, wk_ref[...]).astype(_bf16)
    vb = dot(xv.astype(_bf16), wv_ref[...])

    wl = mv[6:7, :] + dot(jnp.tanh(dot(xw, ww1_ref[...])), ww2_ref[...])
    sp = jnp.maximum(-wl, 0.0) + jnp.log1p(jnp.exp(-jnp.abs(wl)))
    wln_ref[...] = -jnp.exp(-sp - 0.5)

    a_ref[...] = jax.nn.sigmoid(
        mv[7:8, :] + dot(dot(xa, aa1_ref[...]), aa2_ref[...])).astype(_bf16)
    sv = jax.nn.sigmoid(mv[8:9, :] + dot(dot(xv, vv1_ref[...]), vv2_ref[...]))
    v_ref[...] = (vb + (v0_ref[...] - vb) * sv).astype(_bf16)
    g_ref[...] = dot(jax.nn.sigmoid(dot(xg, gg1_ref[...])), gg2_ref[...]).astype(_bf16)


_HG = 4
_GL = _HG * _L


def _k2_body(r_ref, wln_ref, k_ref, v_ref, a_ref,
             kks_ref, kas_ref, rks_ref, gnw_ref, gnb_ref,
             y_ref):
    row = jax.lax.broadcasted_iota(jnp.int32, (_GL, _GL), 0)
    col = jax.lax.broadcasted_iota(jnp.int32, (_GL, _GL), 1)
    same = (row >> 6) == (col >> 6)
    rpos = row & 63
    cpos = col & 63
    m_strict = same & (rpos > cpos)
    m_incl = (same & (rpos >= cpos)).astype(_f32)
    eye_g = (row == col).astype(_f32)
    sel_last = (col == (row | 63)).astype(_f32)
    lrow = jax.lax.broadcasted_iota(jnp.int32, (_GL, _N), 0)
    lcol = jax.lax.broadcasted_iota(jnp.int32, (_GL, _N), 1)
    lane_diag = (lcol == (lrow & 63)).astype(_f32)
    same_f = same.astype(_f32)

    dot = functools.partial(jnp.dot, preferred_element_type=_f32)
    dot_nt = lambda p, q: jax.lax.dot_general(
        p, q, (((1,), (1,)), ((), ())), preferred_element_type=_f32)
    dot_tn = lambda p, q: jax.lax.dot_general(
        p, q, (((0,), (0,)), ((), ())), preferred_element_type=_f32)
    bdiag = lambda x: jnp.concatenate([x] * _HG, axis=1) * same_f

    ones_L = jnp.ones((_L, 1), _f32)
    hv = lambda ref, h: ref[0, :, h * _N:(h + 1) * _N]
    kks_s = jnp.concatenate([ones_L * hv(kks_ref, h) for h in range(_HG)], 0)
    kas_s = jnp.concatenate([ones_L * hv(kas_ref, h) for h in range(_HG)], 0)

    def chunk(i, St):
        sl = pl.ds(pl.multiple_of(i * _L, _L), _L)
        cat = lambda ref: jnp.concatenate(
            [ref[0, sl, h * _N:(h + 1) * _N] for h in range(_HG)], axis=0)
        rc = cat(r_ref).astype(_f32)
        wc = cat(wln_ref)
        kraw = cat(k_ref).astype(_f32)
        vc = cat(v_ref).astype(_f32)
        ac = cat(a_ref).astype(_f32)

        kkc = kraw * kks_s
        ss = jnp.sum(kkc * kkc, axis=-1, keepdims=True)
        kkc = kkc * jax.lax.rsqrt(jnp.maximum(ss, _EPS2))
        kfc = kraw * (1.0 + (ac - 1.0) * kas_s)
        bc = kkc * ac
        anc = -kkc

        clog = dot(m_incl, wc)
        c_in = jnp.exp(clog)
        inv_c = jnp.exp(-clog)
        at = anc * jnp.exp(clog - wc)
        bt = bc * inv_c
        kt = kfc * inv_c
        rt = rc * c_in

        lhs = jnp.concatenate([at, rt], axis=0)
        rhs = jnp.concatenate([bt, kt], axis=0)
        G = dot_nt(lhs, rhs)
        m1 = jnp.where(m_strict, G[:_GL, :_GL], 0.0)
        m2 = jnp.where(m_strict, G[:_GL, _GL:], 0.0)
        rb = G[_GL:, :_GL] * m_incl
        rk = G[_GL:, _GL:] * m_incl

        Nm = m1
        P = eye_g + m1
        for _ in range(5):
            Nm = dot(Nm, Nm)
            P = P + dot(Nm, P)

        zrhs = dot(jnp.concatenate([m2, bdiag(at)], axis=1),
                   jnp.concatenate([vc, St], axis=0))
        Z = dot(P, zrhs)

        O = dot(jnp.concatenate([rb, bdiag(rt), rk], axis=1),
                jnp.concatenate([Z, St, vc], axis=0))
        for h in range(_HG):
            y_ref[0, sl, h * _N:(h + 1) * _N] = O[h * _L:(h + 1) * _L, :].astype(_bf16)

        delta = dot_tn(jnp.concatenate([bdiag(bt), bdiag(kt)], axis=0),
                       jnp.concatenate([Z, vc], axis=0))
        scale = jnp.sum(dot(sel_last, c_in) * lane_diag, axis=-1, keepdims=True)
        return (St + delta) * scale

    jax.lax.fori_loop(0, _NC, chunk, jnp.zeros((_GL, _N), _f32))

    for h in range(_HG):
        hs = slice(h * _N, (h + 1) * _N)
        kas = hv(kas_ref, h)
        rks = hv(rks_ref, h)
        y = y_ref[0, :, hs].astype(_f32)
        mu = jnp.mean(y, axis=-1, keepdims=True)
        yc = y - mu
        var = jnp.mean(yc * yc, axis=-1, keepdims=True)
        yn = yc * jax.lax.rsqrt(var + _GN_EPS) * hv(gnw_ref, h) + hv(gnb_ref, h)

        r = r_ref[0, :, hs].astype(_f32)
        kraw = k_ref[0, :, hs].astype(_f32)
        ac = a_ref[0, :, hs].astype(_f32)
        kf = kraw * (1.0 + (ac - 1.0) * kas)
        bonus = jnp.sum(r * kf * rks, axis=-1, keepdims=True) * v_ref[0, :, hs].astype(_f32)
        y_ref[0, :, hs] = (yn + bonus).astype(_bf16)


def _k3_body(y_ref, g_ref, wo_ref, o_ref):
    yg = y_ref[...] * g_ref[...]
    o_ref[...] = jnp.dot(yg, wo_ref[...], preferred_element_type=_f32)


def kernel(x, v0, xx_r, xx_w, xx_k, xx_v, xx_a, xx_g, ww_b, ww_w1, ww_w2,
           aa_b, aa_w1, aa_w2, vv_b, vv_w1, vv_w2, gg_w1, gg_w2,
           kk_s, ka_s, rk_s, W_r, W_k, W_v, W_o, gn_w, gn_b):
    xs = jnp.pad(x, ((0, 0), (1, 0), (0, 0)))[:, :_T, :]
    x2 = x.reshape(_BT, _C)
    dx2 = (xs - x).reshape(_BT, _C)
    v02 = v0.reshape(_BT, _C)

    mix = jnp.concatenate(
        [t.reshape(1, _C) for t in (xx_r, xx_w, xx_k, xx_v, xx_a, xx_g,
                                    ww_b, aa_b, vv_b)]
        + [jnp.zeros((7, _C), _f32)], axis=0)

    row_spec = pl.BlockSpec((_MB1, _C), lambda i: (i, 0))
    full = lambda s: pl.BlockSpec(s, lambda i: tuple(0 for _ in s))

    grid1 = _BT // _MB1
    r2, wln2, k2, v2, a2, g2 = pl.pallas_call(
        _k1_body,
        grid=(grid1,),
        in_specs=[row_spec, row_spec, row_spec, full((16, _C)),
                  full((_C, _C)), full((_C, _C)), full((_C, _C)),
                  full(ww_w1.shape), full(ww_w2.shape),
                  full(aa_w1.shape), full(aa_w2.shape),
                  full(vv_w1.shape), full(vv_w2.shape),
                  full(gg_w1.shape), full(gg_w2.shape)],
        out_specs=[row_spec] * 6,
        out_shape=[jax.ShapeDtypeStruct((_BT, _C), _bf16),
                   jax.ShapeDtypeStruct((_BT, _C), _f32),
                   jax.ShapeDtypeStruct((_BT, _C), _bf16),
                   jax.ShapeDtypeStruct((_BT, _C), _bf16),
                   jax.ShapeDtypeStruct((_BT, _C), _bf16),
                   jax.ShapeDtypeStruct((_BT, _C), _bf16)],
        compiler_params=pltpu.CompilerParams(
            dimension_semantics=("parallel",),
            vmem_limit_bytes=64 * 1024 * 1024),
    )(x2, dx2, v02, mix,
      W_r.astype(_bf16), W_k.astype(_bf16), W_v.astype(_bf16),
      ww_w1, ww_w2, aa_w1, aa_w2, vv_w1, vv_w2, gg_w1, gg_w2)

    bt3 = lambda t: t.reshape(_B, _T, _C)
    hvec = lambda t: t.reshape(_H // _HG, 1, _HG * _N)

    ng = _H // _HG
    gw = _HG * _N
    head_spec = pl.BlockSpec((1, _T, gw), lambda i: (i // ng, 0, i % ng))
    hvec_spec = pl.BlockSpec((1, 1, gw), lambda i: (i % ng, 0, 0))

    yH = pl.pallas_call(
        _k2_body,
        grid=(_B * ng,),
        in_specs=[head_spec] * 5 + [hvec_spec] * 5,
        out_specs=head_spec,
        out_shape=jax.ShapeDtypeStruct((_B, _T, _C), _bf16),
        compiler_params=pltpu.CompilerParams(
            dimension_semantics=("parallel",),
            vmem_limit_bytes=64 * 1024 * 1024),
    )(bt3(r2), bt3(wln2), bt3(k2), bt3(v2), bt3(a2),
      hvec(kk_s), hvec(ka_s), hvec(rk_s), hvec(gn_w), hvec(gn_b))

    yF = yH.reshape(_BT, _C)

    row3 = pl.BlockSpec((_MB3, _C), lambda i: (i, 0))
    out2 = pl.pallas_call(
        _k3_body,
        grid=(_BT // _MB3,),
        in_specs=[row3, row3, full((_C, _C))],
        out_specs=row3,
        out_shape=jax.ShapeDtypeStruct((_BT, _C), _f32),
        compiler_params=pltpu.CompilerParams(
            dimension_semantics=("parallel",),
            vmem_limit_bytes=64 * 1024 * 1024),
    )(yF, g2, W_o.astype(_bf16))

    return out2.reshape(_B, _T, _C), v0


# speedup vs baseline: 7.7240x; 1.0275x over previous
import functools

import jax
import jax.numpy as jnp
from jax.experimental import pallas as pl
from jax.experimental.pallas import tpu as pltpu

_B, _T, _C = 2, 1024, 2048
_N = 64
_H = _C // _N
_L = 64
_NC = _T // _L
_GN_EPS = 0.00064
_EPS2 = 1e-24
_BT = _B * _T

_MB1 = 128
_MB3 = 256

_f32 = jnp.float32
_bf16 = jnp.bfloat16


def _k1_body(x_ref, dx_ref, v0_ref, mix_ref,
             wr_ref, wk_ref, wv_ref,
             ww1_ref, ww2_ref, aa1_ref, aa2_ref,
             vv1_ref, vv2_ref, gg1_ref, gg2_ref,
             r_ref, wln_ref, k_ref, v_ref, a_ref, g_ref):
    x = x_ref[...]
    dx = dx_ref[...]
    mv = mix_ref[...]
    xr = (x + dx * mv[0:1, :]).astype(_bf16)
    xw = x + dx * mv[1:2, :]
    xk = (x + dx * mv[2:3, :]).astype(_bf16)
    xv = x + dx * mv[3:4, :]
    xa = x + dx * mv[4:5, :]
    xg = x + dx * mv[5:6, :]

    dot = functools.partial(jnp.dot, preferred_element_type=_f32)

    r_ref[...] = dot(xr, wr_ref[...]).astype(_bf16)
    k_ref[...] = dot(xk, wk_ref[...]).astype(_bf16)
    vb = dot(xv.astype(_bf16), wv_ref[...])

    wl = mv[6:7, :] + dot(jnp.tanh(dot(xw, ww1_ref[...])), ww2_ref[...])
    sp = jnp.maximum(-wl, 0.0) + jnp.log1p(jnp.exp(-jnp.abs(wl)))
    wln_ref[...] = -jnp.exp(-sp - 0.5)

    a_ref[...] = jax.nn.sigmoid(
        mv[7:8, :] + dot(dot(xa, aa1_ref[...]), aa2_ref[...])).astype(_bf16)
    sv = jax.nn.sigmoid(mv[8:9, :] + dot(dot(xv, vv1_ref[...]), vv2_ref[...]))
    v_ref[...] = (vb + (v0_ref[...] - vb) * sv).astype(_bf16)
    g_ref[...] = dot(jax.nn.sigmoid(dot(xg, gg1_ref[...])), gg2_ref[...]).astype(_bf16)


_HG = 4
_GL = _HG * _L


def _k2_body(r_ref, wln_ref, k_ref, v_ref, a_ref,
             kks_ref, kas_ref, rks_ref, gnw_ref, gnb_ref,
             y_ref):
    row = jax.lax.broadcasted_iota(jnp.int32, (_GL, _GL), 0)
    col = jax.lax.broadcasted_iota(jnp.int32, (_GL, _GL), 1)
    same = (row >> 6) == (col >> 6)
    rpos = row & 63
    cpos = col & 63
    m_strict = same & (rpos > cpos)
    m_incl = (same & (rpos >= cpos)).astype(_f32)
    eye_g = (row == col).astype(_f32)
    sel_last = (col == (row | 63)).astype(_f32)
    lrow = jax.lax.broadcasted_iota(jnp.int32, (_GL, _N), 0)
    lcol = jax.lax.broadcasted_iota(jnp.int32, (_GL, _N), 1)
    lane_diag = (lcol == (lrow & 63)).astype(_f32)
    same_f = same.astype(_f32)

    dot = functools.partial(jnp.dot, preferred_element_type=_f32)
    dot_nt = lambda p, q: jax.lax.dot_general(
        p, q, (((1,), (1,)), ((), ())), preferred_element_type=_f32)
    dot_tn = lambda p, q: jax.lax.dot_general(
        p, q, (((0,), (0,)), ((), ())), preferred_element_type=_f32)
    wide = lambda x: jnp.concatenate(
        [x[h * _L:(h + 1) * _L, :] for h in range(_HG)], axis=1)
    ext = lambda m: jnp.concatenate(
        [m[h * _L:(h + 1) * _L, h * _N:(h + 1) * _N] for h in range(_HG)], axis=0)

    ones_L = jnp.ones((_L, 1), _f32)
    hv = lambda ref, h: ref[0, :, h * _N:(h + 1) * _N]
    kks_s = jnp.concatenate([ones_L * hv(kks_ref, h) for h in range(_HG)], 0)
    kas_s = jnp.concatenate([ones_L * hv(kas_ref, h) for h in range(_HG)], 0)

    def chunk(i, St):
        sl = pl.ds(pl.multiple_of(i * _L, _L), _L)
        cat = lambda ref: jnp.concatenate(
            [ref[0, sl, h * _N:(h + 1) * _N] for h in range(_HG)], axis=0)
        rc = cat(r_ref).astype(_f32)
        wc = cat(wln_ref)
        kraw = cat(k_ref).astype(_f32)
        vc = cat(v_ref).astype(_f32)
        ac = cat(a_ref).astype(_f32)

        kkc = kraw * kks_s
        ss = jnp.sum(kkc * kkc, axis=-1, keepdims=True)
        kkc = kkc * jax.lax.rsqrt(jnp.maximum(ss, _EPS2))
        kfc = kraw * (1.0 + (ac - 1.0) * kas_s)
        bc = kkc * ac
        anc = -kkc

        clog = dot(m_incl, wc)
        c_in = jnp.exp(clog)
        inv_c = jnp.exp(-clog)
        at = anc * jnp.exp(clog - wc)
        bt = bc * inv_c
        kt = kfc * inv_c
        rt = rc * c_in

        m1 = jnp.where(m_strict, dot_nt(at, bt), 0.0)
        m2 = jnp.where(m_strict, dot_nt(at, kt), 0.0)
        rb = dot_nt(rt, bt) * m_incl
        rk = dot_nt(rt, kt) * m_incl

        Nm = m1
        P = eye_g + m1
        for _ in range(5):
            Nm = dot(Nm, Nm)
            P = P + dot(Nm, P)

        Stw = wide(St)
        atSt = ext(dot(at, Stw))
        rtSt = ext(dot(rt, Stw))
        Z = dot(P, dot(m2, vc) + atSt)

        O = dot(rb, Z) + dot(rk, vc) + rtSt
        for h in range(_HG):
            y_ref[0, sl, h * _N:(h + 1) * _N] = O[h * _L:(h + 1) * _L, :].astype(_bf16)

        dbt = ext(dot_tn(wide(bt), wide(Z)))
        dkt = ext(dot_tn(wide(kt), wide(vc)))
        scale = jnp.sum(dot(sel_last, c_in) * lane_diag, axis=-1, keepdims=True)
        return (St + dbt + dkt) * scale

    jax.lax.fori_loop(0, _NC, chunk, jnp.zeros((_GL, _N), _f32))

    for h in range(_HG):
        hs = slice(h * _N, (h + 1) * _N)
        kas = hv(kas_ref, h)
        rks = hv(rks_ref, h)
        y = y_ref[0, :, hs].astype(_f32)
        mu = jnp.mean(y, axis=-1, keepdims=True)
        yc = y - mu
        var = jnp.mean(yc * yc, axis=-1, keepdims=True)
        yn = yc * jax.lax.rsqrt(var + _GN_EPS) * hv(gnw_ref, h) + hv(gnb_ref, h)

        r = r_ref[0, :, hs].astype(_f32)
        kraw = k_ref[0, :, hs].astype(_f32)
        ac = a_ref[0, :, hs].astype(_f32)
        kf = kraw * (1.0 + (ac - 1.0) * kas)
        bonus = jnp.sum(r * kf * rks, axis=-1, keepdims=True) * v_ref[0, :, hs].astype(_f32)
        y_ref[0, :, hs] = (yn + bonus).astype(_bf16)


def _k3_body(y_ref, g_ref, wo_ref, o_ref):
    yg = y_ref[...] * g_ref[...]
    o_ref[...] = jnp.dot(yg, wo_ref[...], preferred_element_type=_f32)


def kernel(x, v0, xx_r, xx_w, xx_k, xx_v, xx_a, xx_g, ww_b, ww_w1, ww_w2,
           aa_b, aa_w1, aa_w2, vv_b, vv_w1, vv_w2, gg_w1, gg_w2,
           kk_s, ka_s, rk_s, W_r, W_k, W_v, W_o, gn_w, gn_b):
    xs = jnp.pad(x, ((0, 0), (1, 0), (0, 0)))[:, :_T, :]
    x2 = x.reshape(_BT, _C)
    dx2 = (xs - x).reshape(_BT, _C)
    v02 = v0.reshape(_BT, _C)

    mix = jnp.concatenate(
        [t.reshape(1, _C) for t in (xx_r, xx_w, xx_k, xx_v, xx_a, xx_g,
                                    ww_b, aa_b, vv_b)]
        + [jnp.zeros((7, _C), _f32)], axis=0)

    row_spec = pl.BlockSpec((_MB1, _C), lambda i: (i, 0))
    full = lambda s: pl.BlockSpec(s, lambda i: tuple(0 for _ in s))

    grid1 = _BT // _MB1
    r2, wln2, k2, v2, a2, g2 = pl.pallas_call(
        _k1_body,
        grid=(grid1,),
        in_specs=[row_spec, row_spec, row_spec, full((16, _C)),
                  full((_C, _C)), full((_C, _C)), full((_C, _C)),
                  full(ww_w1.shape), full(ww_w2.shape),
                  full(aa_w1.shape), full(aa_w2.shape),
                  full(vv_w1.shape), full(vv_w2.shape),
                  full(gg_w1.shape), full(gg_w2.shape)],
        out_specs=[row_spec] * 6,
        out_shape=[jax.ShapeDtypeStruct((_BT, _C), _bf16),
                   jax.ShapeDtypeStruct((_BT, _C), _f32),
                   jax.ShapeDtypeStruct((_BT, _C), _bf16),
                   jax.ShapeDtypeStruct((_BT, _C), _bf16),
                   jax.ShapeDtypeStruct((_BT, _C), _bf16),
                   jax.ShapeDtypeStruct((_BT, _C), _bf16)],
        compiler_params=pltpu.CompilerParams(
            dimension_semantics=("parallel",),
            vmem_limit_bytes=64 * 1024 * 1024),
    )(x2, dx2, v02, mix,
      W_r.astype(_bf16), W_k.astype(_bf16), W_v.astype(_bf16),
      ww_w1, ww_w2, aa_w1, aa_w2, vv_w1, vv_w2, gg_w1, gg_w2)

    bt3 = lambda t: t.reshape(_B, _T, _C)
    hvec = lambda t: t.reshape(_H // _HG, 1, _HG * _N)

    ng = _H // _HG
    gw = _HG * _N
    head_spec = pl.BlockSpec((1, _T, gw), lambda i: (i // ng, 0, i % ng))
    hvec_spec = pl.BlockSpec((1, 1, gw), lambda i: (i % ng, 0, 0))

    yH = pl.pallas_call(
        _k2_body,
        grid=(_B * ng,),
        in_specs=[head_spec] * 5 + [hvec_spec] * 5,
        out_specs=head_spec,
        out_shape=jax.ShapeDtypeStruct((_B, _T, _C), _bf16),
        compiler_params=pltpu.CompilerParams(
            dimension_semantics=("parallel",),
            vmem_limit_bytes=64 * 1024 * 1024),
    )(bt3(r2), bt3(wln2), bt3(k2), bt3(v2), bt3(a2),
      hvec(kk_s), hvec(ka_s), hvec(rk_s), hvec(gn_w), hvec(gn_b))

    yF = yH.reshape(_BT, _C)

    row3 = pl.BlockSpec((_MB3, _C), lambda i: (i, 0))
    out2 = pl.pallas_call(
        _k3_body,
        grid=(_BT // _MB3,),
        in_specs=[row3, row3, full((_C, _C))],
        out_specs=row3,
        out_shape=jax.ShapeDtypeStruct((_BT, _C), _f32),
        compiler_params=pltpu.CompilerParams(
            dimension_semantics=("parallel",),
            vmem_limit_bytes=64 * 1024 * 1024),
    )(yF, g2, W_o.astype(_bf16))

    return out2.reshape(_B, _T, _C), v0


# speedup vs baseline: 8.0101x; 1.0370x over previous
import functools

import jax
import jax.numpy as jnp
from jax.experimental import pallas as pl
from jax.experimental.pallas import tpu as pltpu

_B, _T, _C = 2, 1024, 2048
_N = 64
_H = _C // _N
_L = 64
_NC = _T // _L
_GN_EPS = 0.00064
_EPS2 = 1e-24
_BT = _B * _T

_MB1 = 128
_MB3 = 256

_f32 = jnp.float32
_bf16 = jnp.bfloat16


def _k1_body(x_ref, dx_ref, v0_ref, mix_ref,
             wr_ref, wk_ref, wv_ref,
             ww1_ref, ww2_ref, aa1_ref, aa2_ref,
             vv1_ref, vv2_ref, gg1_ref, gg2_ref,
             r_ref, wln_ref, k_ref, v_ref, a_ref, g_ref):
    x = x_ref[...]
    dx = dx_ref[...]
    mv = mix_ref[...]
    xr = (x + dx * mv[0:1, :]).astype(_bf16)
    xw = x + dx * mv[1:2, :]
    xk = (x + dx * mv[2:3, :]).astype(_bf16)
    xv = x + dx * mv[3:4, :]
    xa = x + dx * mv[4:5, :]
    xg = x + dx * mv[5:6, :]

    dot = functools.partial(jnp.dot, preferred_element_type=_f32)

    r_ref[...] = dot(xr, wr_ref[...]).astype(_bf16)
    k_ref[...] = dot(xk, wk_ref[...]).astype(_bf16)
    vb = dot(xv.astype(_bf16), wv_ref[...])

    wl = mv[6:7, :] + dot(jnp.tanh(dot(xw, ww1_ref[...])), ww2_ref[...])
    sp = jnp.maximum(-wl, 0.0) + jnp.log1p(jnp.exp(-jnp.abs(wl)))
    wln_ref[...] = -jnp.exp(-sp - 0.5)

    a_ref[...] = jax.nn.sigmoid(
        mv[7:8, :] + dot(dot(xa, aa1_ref[...]), aa2_ref[...])).astype(_bf16)
    sv = jax.nn.sigmoid(mv[8:9, :] + dot(dot(xv, vv1_ref[...]), vv2_ref[...]))
    v_ref[...] = (vb + (v0_ref[...] - vb) * sv).astype(_bf16)
    g_ref[...] = dot(jax.nn.sigmoid(dot(xg, gg1_ref[...])), gg2_ref[...]).astype(_bf16)


_HG = 4
_GL = _HG * _L


def _k2_body(r_ref, wln_ref, k_ref, v_ref, a_ref,
             kks_ref, kas_ref, rks_ref, gnw_ref, gnb_ref,
             y_ref):
    row = jax.lax.broadcasted_iota(jnp.int32, (_GL, _GL), 0)
    col = jax.lax.broadcasted_iota(jnp.int32, (_GL, _GL), 1)
    same = (row >> 6) == (col >> 6)
    rpos = row & 63
    cpos = col & 63
    m_strict = same & (rpos > cpos)
    m_incl = (same & (rpos >= cpos)).astype(_f32)
    eye_g = (row == col).astype(_f32)
    sel_last = (col == (row | 63)).astype(_f32)
    lrow = jax.lax.broadcasted_iota(jnp.int32, (_GL, _N), 0)
    lcol = jax.lax.broadcasted_iota(jnp.int32, (_GL, _N), 1)
    lane_diag = (lcol == (lrow & 63)).astype(_f32)
    same_f = same.astype(_f32)

    dot = functools.partial(jnp.dot, preferred_element_type=_f32)
    dot_nt = lambda p, q: jax.lax.dot_general(
        p, q, (((1,), (1,)), ((), ())), preferred_element_type=_f32)
    dot_tn = lambda p, q: jax.lax.dot_general(
        p, q, (((0,), (0,)), ((), ())), preferred_element_type=_f32)
    wide = lambda x: jnp.concatenate(
        [x[h * _L:(h + 1) * _L, :] for h in range(_HG)], axis=1)
    ext = lambda m: jnp.concatenate(
        [m[h * _L:(h + 1) * _L, h * _N:(h + 1) * _N] for h in range(_HG)], axis=0)

    ones_L = jnp.ones((_L, 1), _f32)
    hv = lambda ref, h: ref[0, :, h * _N:(h + 1) * _N]
    kks_s = jnp.concatenate([ones_L * hv(kks_ref, h) for h in range(_HG)], 0)
    kas_s = jnp.concatenate([ones_L * hv(kas_ref, h) for h in range(_HG)], 0)

    def chunk(i, St):
        sl = pl.ds(pl.multiple_of(i * _L, _L), _L)
        cat = lambda ref: jnp.concatenate(
            [ref[0, sl, h * _N:(h + 1) * _N] for h in range(_HG)], axis=0)
        rc = cat(r_ref).astype(_f32)
        wc = cat(wln_ref)
        kraw = cat(k_ref).astype(_f32)
        vc = cat(v_ref).astype(_f32)
        ac = cat(a_ref).astype(_f32)

        kkc = kraw * kks_s
        ss = jnp.sum(kkc * kkc, axis=-1, keepdims=True)
        kkc = kkc * jax.lax.rsqrt(jnp.maximum(ss, _EPS2))
        kfc = kraw * (1.0 + (ac - 1.0) * kas_s)
        bc = kkc * ac
        anc = -kkc

        clog = dot(m_incl, wc)
        c_in = jnp.exp(clog)
        inv_c = jnp.exp(-clog)
        at = anc * jnp.exp(clog - wc)
        bt = bc * inv_c
        kt = kfc * inv_c
        rt = rc * c_in

        m1 = jnp.where(m_strict, dot_nt(at, bt), 0.0)
        m2 = jnp.where(m_strict, dot_nt(at, kt), 0.0)
        rb = dot_nt(rt, bt) * m_incl
        rk = dot_nt(rt, kt) * m_incl

        Nm = m1
        P = eye_g + m1
        for _ in range(5):
            Nm = dot(Nm, Nm)
            P = P + dot(Nm, P)

        Stw = wide(St)
        atSt = ext(dot(at, Stw))
        rtSt = ext(dot(rt, Stw))
        Z = dot(P, dot(m2, vc) + atSt)

        O = dot(rb, Z) + dot(rk, vc) + rtSt
        for h in range(_HG):
            y_ref[0, sl, h * _N:(h + 1) * _N] = O[h * _L:(h + 1) * _L, :].astype(_bf16)

        dbt = ext(dot_tn(wide(bt), wide(Z)))
        dkt = ext(dot_tn(wide(kt), wide(vc)))
        scale = jnp.sum(dot(sel_last, c_in) * lane_diag, axis=-1, keepdims=True)
        return (St + dbt + dkt) * scale

    jax.lax.fori_loop(0, _NC, chunk, jnp.zeros((_GL, _N), _f32))

    gsum = (row >> 6 == col >> 6).astype(_f32)
    gmean = gsum * (1.0 / _N)
    y = y_ref[0, :, :].astype(_f32)
    mu = jnp.dot(y, gmean, preferred_element_type=_f32)
    ms = jnp.dot(y * y, gmean, preferred_element_type=_f32)
    var = ms - mu * mu
    yn = ((y - mu) * jax.lax.rsqrt(var + _GN_EPS) * gnw_ref[0, :, :]
          + gnb_ref[0, :, :])

    r = r_ref[0, :, :].astype(_f32)
    kraw = k_ref[0, :, :].astype(_f32)
    ac = a_ref[0, :, :].astype(_f32)
    kf = kraw * (1.0 + (ac - 1.0) * kas_ref[0, :, :])
    s = jnp.dot(r * kf * rks_ref[0, :, :], gsum, preferred_element_type=_f32)
    y_ref[0, :, :] = (yn + s * v_ref[0, :, :].astype(_f32)).astype(_bf16)


def _k3_body(y_ref, g_ref, wo_ref, o_ref):
    yg = y_ref[...] * g_ref[...]
    o_ref[...] = jnp.dot(yg, wo_ref[...], preferred_element_type=_f32)


def kernel(x, v0, xx_r, xx_w, xx_k, xx_v, xx_a, xx_g, ww_b, ww_w1, ww_w2,
           aa_b, aa_w1, aa_w2, vv_b, vv_w1, vv_w2, gg_w1, gg_w2,
           kk_s, ka_s, rk_s, W_r, W_k, W_v, W_o, gn_w, gn_b):
    xs = jnp.pad(x, ((0, 0), (1, 0), (0, 0)))[:, :_T, :]
    x2 = x.reshape(_BT, _C)
    dx2 = (xs - x).reshape(_BT, _C)
    v02 = v0.reshape(_BT, _C)

    mix = jnp.concatenate(
        [t.reshape(1, _C) for t in (xx_r, xx_w, xx_k, xx_v, xx_a, xx_g,
                                    ww_b, aa_b, vv_b)]
        + [jnp.zeros((7, _C), _f32)], axis=0)

    row_spec = pl.BlockSpec((_MB1, _C), lambda i: (i, 0))
    full = lambda s: pl.BlockSpec(s, lambda i: tuple(0 for _ in s))

    grid1 = _BT // _MB1
    r2, wln2, k2, v2, a2, g2 = pl.pallas_call(
        _k1_body,
        grid=(grid1,),
        in_specs=[row_spec, row_spec, row_spec, full((16, _C)),
                  full((_C, _C)), full((_C, _C)), full((_C, _C)),
                  full(ww_w1.shape), full(ww_w2.shape),
                  full(aa_w1.shape), full(aa_w2.shape),
                  full(vv_w1.shape), full(vv_w2.shape),
                  full(gg_w1.shape), full(gg_w2.shape)],
        out_specs=[row_spec] * 6,
        out_shape=[jax.ShapeDtypeStruct((_BT, _C), _bf16),
                   jax.ShapeDtypeStruct((_BT, _C), _f32),
                   jax.ShapeDtypeStruct((_BT, _C), _bf16),
                   jax.ShapeDtypeStruct((_BT, _C), _bf16),
                   jax.ShapeDtypeStruct((_BT, _C), _bf16),
                   jax.ShapeDtypeStruct((_BT, _C), _bf16)],
        compiler_params=pltpu.CompilerParams(
            dimension_semantics=("parallel",),
            vmem_limit_bytes=64 * 1024 * 1024),
    )(x2, dx2, v02, mix,
      W_r.astype(_bf16), W_k.astype(_bf16), W_v.astype(_bf16),
      ww_w1, ww_w2, aa_w1, aa_w2, vv_w1, vv_w2, gg_w1, gg_w2)

    bt3 = lambda t: t.reshape(_B, _T, _C)
    hvec = lambda t: t.reshape(_H // _HG, 1, _HG * _N)

    ng = _H // _HG
    gw = _HG * _N
    head_spec = pl.BlockSpec((1, _T, gw), lambda i: (i // ng, 0, i % ng))
    hvec_spec = pl.BlockSpec((1, 1, gw), lambda i: (i % ng, 0, 0))

    yH = pl.pallas_call(
        _k2_body,
        grid=(_B * ng,),
        in_specs=[head_spec] * 5 + [hvec_spec] * 5,
        out_specs=head_spec,
        out_shape=jax.ShapeDtypeStruct((_B, _T, _C), _bf16),
        compiler_params=pltpu.CompilerParams(
            dimension_semantics=("parallel",),
            vmem_limit_bytes=64 * 1024 * 1024),
    )(bt3(r2), bt3(wln2), bt3(k2), bt3(v2), bt3(a2),
      hvec(kk_s), hvec(ka_s), hvec(rk_s), hvec(gn_w), hvec(gn_b))

    yF = yH.reshape(_BT, _C)

    row3 = pl.BlockSpec((_MB3, _C), lambda i: (i, 0))
    out2 = pl.pallas_call(
        _k3_body,
        grid=(_BT // _MB3,),
        in_specs=[row3, row3, full((_C, _C))],
        out_specs=row3,
        out_shape=jax.ShapeDtypeStruct((_BT, _C), _f32),
        compiler_params=pltpu.CompilerParams(
            dimension_semantics=("parallel",),
            vmem_limit_bytes=64 * 1024 * 1024),
    )(yF, g2, W_o.astype(_bf16))

    return out2.reshape(_B, _T, _C), v0


# speedup vs baseline: 8.1071x; 1.0121x over previous
import functools

import jax
import jax.numpy as jnp
from jax.experimental import pallas as pl
from jax.experimental.pallas import tpu as pltpu

_B, _T, _C = 2, 1024, 2048
_N = 64
_H = _C // _N
_L = 64
_NC = _T // _L
_GN_EPS = 0.00064
_EPS2 = 1e-24
_BT = _B * _T

_MB1 = 256
_MB3 = 256

_f32 = jnp.float32
_bf16 = jnp.bfloat16


def _k1_body(x_ref, dx_ref, v0_ref, mix_ref,
             wr_ref, wk_ref, wv_ref,
             ww1_ref, ww2_ref, aa1_ref, aa2_ref,
             vv1_ref, vv2_ref, gg1_ref, gg2_ref,
             r_ref, wln_ref, k_ref, v_ref, a_ref, g_ref):
    x = x_ref[...]
    dx = dx_ref[...]
    mv = mix_ref[...]
    xr = (x + dx * mv[0:1, :]).astype(_bf16)
    xw = x + dx * mv[1:2, :]
    xk = (x + dx * mv[2:3, :]).astype(_bf16)
    xv = x + dx * mv[3:4, :]
    xa = x + dx * mv[4:5, :]
    xg = x + dx * mv[5:6, :]

    dot = functools.partial(jnp.dot, preferred_element_type=_f32)

    r_ref[...] = dot(xr, wr_ref[...]).astype(_bf16)
    k_ref[...] = dot(xk, wk_ref[...]).astype(_bf16)
    vb = dot(xv.astype(_bf16), wv_ref[...])

    wl = mv[6:7, :] + dot(jnp.tanh(dot(xw, ww1_ref[...])), ww2_ref[...])
    sp = jnp.maximum(-wl, 0.0) + jnp.log1p(jnp.exp(-jnp.abs(wl)))
    wln_ref[...] = -jnp.exp(-sp - 0.5)

    a_ref[...] = jax.nn.sigmoid(
        mv[7:8, :] + dot(dot(xa, aa1_ref[...]), aa2_ref[...])).astype(_bf16)
    sv = jax.nn.sigmoid(mv[8:9, :] + dot(dot(xv, vv1_ref[...]), vv2_ref[...]))
    v_ref[...] = (vb + (v0_ref[...] - vb) * sv).astype(_bf16)
    g_ref[...] = dot(jax.nn.sigmoid(dot(xg, gg1_ref[...])), gg2_ref[...]).astype(_bf16)


_HG = 4
_GL = _HG * _L


def _k2_body(r_ref, wln_ref, k_ref, v_ref, a_ref,
             kks_ref, kas_ref, rks_ref, gnw_ref, gnb_ref,
             y_ref):
    row = jax.lax.broadcasted_iota(jnp.int32, (_GL, _GL), 0)
    col = jax.lax.broadcasted_iota(jnp.int32, (_GL, _GL), 1)
    same = (row >> 6) == (col >> 6)
    rpos = row & 63
    cpos = col & 63
    m_strict = same & (rpos > cpos)
    m_incl = (same & (rpos >= cpos)).astype(_f32)
    eye_g = (row == col).astype(_f32)
    sel_last = (col == (row | 63)).astype(_f32)
    lrow = jax.lax.broadcasted_iota(jnp.int32, (_GL, _N), 0)
    lcol = jax.lax.broadcasted_iota(jnp.int32, (_GL, _N), 1)
    lane_diag = (lcol == (lrow & 63)).astype(_f32)
    same_f = same.astype(_f32)

    dot = functools.partial(jnp.dot, preferred_element_type=_f32)
    dot_nt = lambda p, q: jax.lax.dot_general(
        p, q, (((1,), (1,)), ((), ())), preferred_element_type=_f32)
    dot_tn = lambda p, q: jax.lax.dot_general(
        p, q, (((0,), (0,)), ((), ())), preferred_element_type=_f32)
    wide = lambda x: jnp.concatenate(
        [x[h * _L:(h + 1) * _L, :] for h in range(_HG)], axis=1)
    ext = lambda m: jnp.concatenate(
        [m[h * _L:(h + 1) * _L, h * _N:(h + 1) * _N] for h in range(_HG)], axis=0)

    ones_L = jnp.ones((_L, 1), _f32)
    hv = lambda ref, h: ref[0, :, h * _N:(h + 1) * _N]
    kks_s = jnp.concatenate([ones_L * hv(kks_ref, h) for h in range(_HG)], 0)
    kas_s = jnp.concatenate([ones_L * hv(kas_ref, h) for h in range(_HG)], 0)

    def chunk(i, St):
        sl = pl.ds(pl.multiple_of(i * _L, _L), _L)
        cat = lambda ref: jnp.concatenate(
            [ref[0, sl, h * _N:(h + 1) * _N] for h in range(_HG)], axis=0)
        rc = cat(r_ref).astype(_f32)
        wc = cat(wln_ref)
        kraw = cat(k_ref).astype(_f32)
        vc = cat(v_ref).astype(_f32)
        ac = cat(a_ref).astype(_f32)

        kkc = kraw * kks_s
        ss = jnp.sum(kkc * kkc, axis=-1, keepdims=True)
        kkc = kkc * jax.lax.rsqrt(jnp.maximum(ss, _EPS2))
        kfc = kraw * (1.0 + (ac - 1.0) * kas_s)
        bc = kkc * ac
        anc = -kkc

        clog = dot(m_incl, wc)
        c_in = jnp.exp(clog)
        inv_c = jnp.exp(-clog)
        at = anc * jnp.exp(clog - wc)
        bt = bc * inv_c
        kt = kfc * inv_c
        rt = rc * c_in

        m1 = jnp.where(m_strict, dot_nt(at, bt), 0.0)
        m2 = jnp.where(m_strict, dot_nt(at, kt), 0.0)
        rb = dot_nt(rt, bt) * m_incl
        rk = dot_nt(rt, kt) * m_incl

        Nm = m1
        P = eye_g + m1
        for _ in range(5):
            Nm = dot(Nm, Nm)
            P = P + dot(Nm, P)

        Stw = wide(St)
        atSt = ext(dot(at, Stw))
        rtSt = ext(dot(rt, Stw))
        Z = dot(P, dot(m2, vc) + atSt)

        O = dot(rb, Z) + dot(rk, vc) + rtSt
        for h in range(_HG):
            y_ref[0, sl, h * _N:(h + 1) * _N] = O[h * _L:(h + 1) * _L, :].astype(_bf16)

        dbt = ext(dot_tn(wide(bt), wide(Z)))
        dkt = ext(dot_tn(wide(kt), wide(vc)))
        scale = jnp.sum(dot(sel_last, c_in) * lane_diag, axis=-1, keepdims=True)
        return (St + dbt + dkt) * scale

    jax.lax.fori_loop(0, _NC, chunk, jnp.zeros((_GL, _N), _f32))

    gsum = (row >> 6 == col >> 6).astype(_f32)
    gmean = gsum * (1.0 / _N)
    y = y_ref[0, :, :].astype(_f32)
    mu = jnp.dot(y, gmean, preferred_element_type=_f32)
    ms = jnp.dot(y * y, gmean, preferred_element_type=_f32)
    var = ms - mu * mu
    yn = ((y - mu) * jax.lax.rsqrt(var + _GN_EPS) * gnw_ref[0, :, :]
          + gnb_ref[0, :, :])

    r = r_ref[0, :, :].astype(_f32)
    kraw = k_ref[0, :, :].astype(_f32)
    ac = a_ref[0, :, :].astype(_f32)
    kf = kraw * (1.0 + (ac - 1.0) * kas_ref[0, :, :])
    s = jnp.dot(r * kf * rks_ref[0, :, :], gsum, preferred_element_type=_f32)
    y_ref[0, :, :] = (yn + s * v_ref[0, :, :].astype(_f32)).astype(_bf16)


def _k3_body(y_ref, g_ref, wo_ref, o_ref):
    yg = y_ref[...] * g_ref[...]
    o_ref[...] = jnp.dot(yg, wo_ref[...], preferred_element_type=_f32)


def kernel(x, v0, xx_r, xx_w, xx_k, xx_v, xx_a, xx_g, ww_b, ww_w1, ww_w2,
           aa_b, aa_w1, aa_w2, vv_b, vv_w1, vv_w2, gg_w1, gg_w2,
           kk_s, ka_s, rk_s, W_r, W_k, W_v, W_o, gn_w, gn_b):
    xs = jnp.pad(x, ((0, 0), (1, 0), (0, 0)))[:, :_T, :]
    x2 = x.reshape(_BT, _C)
    dx2 = (xs - x).reshape(_BT, _C)
    v02 = v0.reshape(_BT, _C)

    mix = jnp.concatenate(
        [t.reshape(1, _C) for t in (xx_r, xx_w, xx_k, xx_v, xx_a, xx_g,
                                    ww_b, aa_b, vv_b)]
        + [jnp.zeros((7, _C), _f32)], axis=0)

    row_spec = pl.BlockSpec((_MB1, _C), lambda i: (i, 0))
    full = lambda s: pl.BlockSpec(s, lambda i: tuple(0 for _ in s))

    grid1 = _BT // _MB1
    r2, wln2, k2, v2, a2, g2 = pl.pallas_call(
        _k1_body,
        grid=(grid1,),
        in_specs=[row_spec, row_spec, row_spec, full((16, _C)),
                  full((_C, _C)), full((_C, _C)), full((_C, _C)),
                  full(ww_w1.shape), full(ww_w2.shape),
                  full(aa_w1.shape), full(aa_w2.shape),
                  full(vv_w1.shape), full(vv_w2.shape),
                  full(gg_w1.shape), full(gg_w2.shape)],
        out_specs=[row_spec] * 6,
        out_shape=[jax.ShapeDtypeStruct((_BT, _C), _bf16),
                   jax.ShapeDtypeStruct((_BT, _C), _f32),
                   jax.ShapeDtypeStruct((_BT, _C), _bf16),
                   jax.ShapeDtypeStruct((_BT, _C), _bf16),
                   jax.ShapeDtypeStruct((_BT, _C), _bf16),
                   jax.ShapeDtypeStruct((_BT, _C), _bf16)],
        compiler_params=pltpu.CompilerParams(
            dimension_semantics=("parallel",),
            vmem_limit_bytes=64 * 1024 * 1024),
    )(x2, dx2, v02, mix,
      W_r.astype(_bf16), W_k.astype(_bf16), W_v.astype(_bf16),
      ww_w1, ww_w2, aa_w1, aa_w2, vv_w1, vv_w2, gg_w1, gg_w2)

    bt3 = lambda t: t.reshape(_B, _T, _C)
    hvec = lambda t: t.reshape(_H // _HG, 1, _HG * _N)

    ng = _H // _HG
    gw = _HG * _N
    head_spec = pl.BlockSpec((1, _T, gw), lambda i: (i // ng, 0, i % ng))
    hvec_spec = pl.BlockSpec((1, 1, gw), lambda i: (i % ng, 0, 0))

    yH = pl.pallas_call(
        _k2_body,
        grid=(_B * ng,),
        in_specs=[head_spec] * 5 + [hvec_spec] * 5,
        out_specs=head_spec,
        out_shape=jax.ShapeDtypeStruct((_B, _T, _C), _bf16),
        compiler_params=pltpu.CompilerParams(
            dimension_semantics=("parallel",),
            vmem_limit_bytes=64 * 1024 * 1024),
    )(bt3(r2), bt3(wln2), bt3(k2), bt3(v2), bt3(a2),
      hvec(kk_s), hvec(ka_s), hvec(rk_s), hvec(gn_w), hvec(gn_b))

    yF = yH.reshape(_BT, _C)

    row3 = pl.BlockSpec((_MB3, _C), lambda i: (i, 0))
    out2 = pl.pallas_call(
        _k3_body,
        grid=(_BT // _MB3,),
        in_specs=[row3, row3, full((_C, _C))],
        out_specs=row3,
        out_shape=jax.ShapeDtypeStruct((_BT, _C), _f32),
        compiler_params=pltpu.CompilerParams(
            dimension_semantics=("parallel",),
            vmem_limit_bytes=64 * 1024 * 1024),
    )(yF, g2, W_o.astype(_bf16))

    return out2.reshape(_B, _T, _C), v0


# speedup vs baseline: 8.1258x; 1.0023x over previous
import functools

import jax
import jax.numpy as jnp
from jax.experimental import pallas as pl
from jax.experimental.pallas import tpu as pltpu

_B, _T, _C = 2, 1024, 2048
_N = 64
_H = _C // _N
_L = 64
_NC = _T // _L
_GN_EPS = 0.00064
_EPS2 = 1e-24
_BT = _B * _T

_MB1 = 256
_MB3 = 256

_f32 = jnp.float32
_bf16 = jnp.bfloat16


def _k1_body(x_ref, dx_ref, v0_ref, mix_ref,
             wr_ref, wk_ref, wv_ref,
             ww1_ref, ww2_ref, aa1_ref, aa2_ref,
             vv1_ref, vv2_ref, gg1_ref, gg2_ref,
             r_ref, wln_ref, k_ref, v_ref, a_ref, g_ref):
    x = x_ref[...]
    dx = dx_ref[...]
    mv = mix_ref[...]
    xr = (x + dx * mv[0:1, :]).astype(_bf16)
    xw = x + dx * mv[1:2, :]
    xk = (x + dx * mv[2:3, :]).astype(_bf16)
    xv = x + dx * mv[3:4, :]
    xa = x + dx * mv[4:5, :]
    xg = x + dx * mv[5:6, :]

    dot = functools.partial(jnp.dot, preferred_element_type=_f32)

    r_ref[...] = dot(xr, wr_ref[...]).astype(_bf16)
    k_ref[...] = dot(xk, wk_ref[...]).astype(_bf16)
    vb = dot(xv.astype(_bf16), wv_ref[...])

    wl = mv[6:7, :] + dot(jnp.tanh(dot(xw, ww1_ref[...])), ww2_ref[...])
    sp = jnp.maximum(-wl, 0.0) + jnp.log1p(jnp.exp(-jnp.abs(wl)))
    wln_ref[...] = -jnp.exp(-sp - 0.5)

    a_ref[...] = jax.nn.sigmoid(
        mv[7:8, :] + dot(dot(xa, aa1_ref[...]), aa2_ref[...])).astype(_bf16)
    sv = jax.nn.sigmoid(mv[8:9, :] + dot(dot(xv, vv1_ref[...]), vv2_ref[...]))
    v_ref[...] = (vb + (v0_ref[...] - vb) * sv).astype(_bf16)
    g_ref[...] = dot(jax.nn.sigmoid(dot(xg, gg1_ref[...])), gg2_ref[...]).astype(_bf16)


_HG = 4
_GL = _HG * _L


def _k2_body(r_ref, wln_ref, k_ref, v_ref, a_ref,
             kks_ref, kas_ref, rks_ref, gnw_ref, gnb_ref,
             y_ref):
    row = jax.lax.broadcasted_iota(jnp.int32, (_GL, _GL), 0)
    col = jax.lax.broadcasted_iota(jnp.int32, (_GL, _GL), 1)
    same = (row >> 6) == (col >> 6)
    rpos = row & 63
    cpos = col & 63
    m_strict = same & (rpos > cpos)
    m_incl = (same & (rpos >= cpos)).astype(_f32)
    eye_g = (row == col).astype(_f32)
    sel_last = (col == (row | 63)).astype(_f32)
    lrow = jax.lax.broadcasted_iota(jnp.int32, (_GL, _N), 0)
    lcol = jax.lax.broadcasted_iota(jnp.int32, (_GL, _N), 1)
    lane_diag = (lcol == (lrow & 63)).astype(_f32)
    dot = functools.partial(jnp.dot, preferred_element_type=_f32)
    dot_nt = lambda p, q: jax.lax.dot_general(
        p, q, (((1,), (1,)), ((), ())), preferred_element_type=_f32)
    dot_tn = lambda p, q: jax.lax.dot_general(
        p, q, (((0,), (0,)), ((), ())), preferred_element_type=_f32)
    wide = lambda x: jnp.concatenate(
        [x[h * _L:(h + 1) * _L, :] for h in range(_HG)], axis=1)
    ext = lambda m: jnp.concatenate(
        [m[h * _L:(h + 1) * _L, h * _N:(h + 1) * _N] for h in range(_HG)], axis=0)

    ones_L = jnp.ones((_L, 1), _f32)
    hv = lambda ref, h: ref[0, :, h * _N:(h + 1) * _N]
    kks_s = jnp.concatenate([ones_L * hv(kks_ref, h) for h in range(_HG)], 0)
    kas_s = jnp.concatenate([ones_L * hv(kas_ref, h) for h in range(_HG)], 0)

    def chunk(i, St):
        sl = pl.ds(pl.multiple_of(i * _L, _L), _L)
        cat = lambda ref: jnp.concatenate(
            [ref[0, sl, h * _N:(h + 1) * _N] for h in range(_HG)], axis=0)
        rc = cat(r_ref).astype(_f32)
        wc = cat(wln_ref)
        kraw = cat(k_ref).astype(_f32)
        vc = cat(v_ref).astype(_f32)
        ac = cat(a_ref).astype(_f32)

        kkc = kraw * kks_s
        ss = jnp.sum(kkc * kkc, axis=-1, keepdims=True)
        kkc = kkc * jax.lax.rsqrt(jnp.maximum(ss, _EPS2))
        kfc = kraw * (1.0 + (ac - 1.0) * kas_s)
        bc = kkc * ac
        anc = -kkc

        clog = dot(m_incl, wc)
        c_in = jnp.exp(clog)
        inv_c = jnp.exp(-clog)
        at = anc * jnp.exp(clog - wc)
        bt = bc * inv_c
        kt = kfc * inv_c
        rt = rc * c_in

        m1 = jnp.where(m_strict, dot_nt(at, bt), 0.0)
        m2 = jnp.where(m_strict, dot_nt(at, kt), 0.0)
        rb = dot_nt(rt, bt) * m_incl
        rk = dot_nt(rt, kt) * m_incl

        Nm = m1
        P = eye_g + m1
        for _ in range(5):
            Nm = dot(Nm, Nm)
            P = P + dot(Nm, P)

        Stw = wide(St)
        atSt = ext(dot(at, Stw))
        rtSt = ext(dot(rt, Stw))
        Z = dot(P, dot(m2, vc) + atSt)

        O = dot(rb, Z) + dot(rk, vc) + rtSt
        for h in range(_HG):
            y_ref[0, sl, h * _N:(h + 1) * _N] = O[h * _L:(h + 1) * _L, :].astype(_bf16)

        dbt = ext(dot_tn(wide(bt), wide(Z)))
        dkt = ext(dot_tn(wide(kt), wide(vc)))
        scale = jnp.sum(dot(sel_last, c_in) * lane_diag, axis=-1, keepdims=True)
        return (St + dbt + dkt) * scale

    jax.lax.fori_loop(0, _NC, chunk, jnp.zeros((_GL, _N), _f32))

    gsum = (row >> 6 == col >> 6).astype(_f32)
    gmean = gsum * (1.0 / _N)
    y = y_ref[0, :, :].astype(_f32)
    mu = jnp.dot(y, gmean, preferred_element_type=_f32)
    ms = jnp.dot(y * y, gmean, preferred_element_type=_f32)
    var = ms - mu * mu
    yn = ((y - mu) * jax.lax.rsqrt(var + _GN_EPS) * gnw_ref[0, :, :]
          + gnb_ref[0, :, :])

    r = r_ref[0, :, :].astype(_f32)
    kraw = k_ref[0, :, :].astype(_f32)
    ac = a_ref[0, :, :].astype(_f32)
    kf = kraw * (1.0 + (ac - 1.0) * kas_ref[0, :, :])
    s = jnp.dot(r * kf * rks_ref[0, :, :], gsum, preferred_element_type=_f32)
    y_ref[0, :, :] = (yn + s * v_ref[0, :, :].astype(_f32)).astype(_bf16)


def _k3_body(y_ref, g_ref, wo_ref, o_ref):
    yg = y_ref[...] * g_ref[...]
    o_ref[...] = jnp.dot(yg, wo_ref[...], preferred_element_type=_f32)


def kernel(x, v0, xx_r, xx_w, xx_k, xx_v, xx_a, xx_g, ww_b, ww_w1, ww_w2,
           aa_b, aa_w1, aa_w2, vv_b, vv_w1, vv_w2, gg_w1, gg_w2,
           kk_s, ka_s, rk_s, W_r, W_k, W_v, W_o, gn_w, gn_b):
    xs = jnp.pad(x, ((0, 0), (1, 0), (0, 0)))[:, :_T, :]
    x2 = x.reshape(_BT, _C)
    dx2 = (xs - x).reshape(_BT, _C)
    v02 = v0.reshape(_BT, _C)

    mix = jnp.concatenate(
        [t.reshape(1, _C) for t in (xx_r, xx_w, xx_k, xx_v, xx_a, xx_g,
                                    ww_b, aa_b, vv_b)]
        + [jnp.zeros((7, _C), _f32)], axis=0)

    row_spec = pl.BlockSpec((_MB1, _C), lambda i: (i, 0))
    full = lambda s: pl.BlockSpec(s, lambda i: tuple(0 for _ in s))

    grid1 = _BT // _MB1
    r2, wln2, k2, v2, a2, g2 = pl.pallas_call(
        _k1_body,
        grid=(grid1,),
        in_specs=[row_spec, row_spec, row_spec, full((16, _C)),
                  full((_C, _C)), full((_C, _C)), full((_C, _C)),
                  full(ww_w1.shape), full(ww_w2.shape),
                  full(aa_w1.shape), full(aa_w2.shape),
                  full(vv_w1.shape), full(vv_w2.shape),
                  full(gg_w1.shape), full(gg_w2.shape)],
        out_specs=[row_spec] * 6,
        out_shape=[jax.ShapeDtypeStruct((_BT, _C), _bf16),
                   jax.ShapeDtypeStruct((_BT, _C), _f32),
                   jax.ShapeDtypeStruct((_BT, _C), _bf16),
                   jax.ShapeDtypeStruct((_BT, _C), _bf16),
                   jax.ShapeDtypeStruct((_BT, _C), _bf16),
                   jax.ShapeDtypeStruct((_BT, _C), _bf16)],
        compiler_params=pltpu.CompilerParams(
            dimension_semantics=("parallel",),
            vmem_limit_bytes=64 * 1024 * 1024),
    )(x2, dx2, v02, mix,
      W_r.astype(_bf16), W_k.astype(_bf16), W_v.astype(_bf16),
      ww_w1, ww_w2, aa_w1, aa_w2, vv_w1, vv_w2, gg_w1, gg_w2)

    bt3 = lambda t: t.reshape(_B, _T, _C)
    hvec = lambda t: t.reshape(_H // _HG, 1, _HG * _N)

    ng = _H // _HG
    gw = _HG * _N
    head_spec = pl.BlockSpec((1, _T, gw), lambda i: (i // ng, 0, i % ng))
    hvec_spec = pl.BlockSpec((1, 1, gw), lambda i: (i % ng, 0, 0))

    yH = pl.pallas_call(
        _k2_body,
        grid=(_B * ng,),
        in_specs=[head_spec] * 5 + [hvec_spec] * 5,
        out_specs=head_spec,
        out_shape=jax.ShapeDtypeStruct((_B, _T, _C), _bf16),
        compiler_params=pltpu.CompilerParams(
            dimension_semantics=("parallel",),
            vmem_limit_bytes=64 * 1024 * 1024),
    )(bt3(r2), bt3(wln2), bt3(k2), bt3(v2), bt3(a2),
      hvec(kk_s), hvec(ka_s), hvec(rk_s), hvec(gn_w), hvec(gn_b))

    yF = yH.reshape(_BT, _C)

    row3 = pl.BlockSpec((_MB3, _C), lambda i: (i, 0))
    out2 = pl.pallas_call(
        _k3_body,
        grid=(_BT // _MB3,),
        in_specs=[row3, row3, full((_C, _C))],
        out_specs=row3,
        out_shape=jax.ShapeDtypeStruct((_BT, _C), _f32),
        compiler_params=pltpu.CompilerParams(
            dimension_semantics=("parallel",),
            vmem_limit_bytes=64 * 1024 * 1024),
    )(yF, g2, W_o.astype(_bf16))

    return out2.reshape(_B, _T, _C), v0
